# baseline (speedup 1.0000x reference)
import functools

import jax
import jax.numpy as jnp
from jax import lax
from jax.experimental import pallas as pl
from jax.experimental.pallas import tpu as pltpu
from jax.experimental.pallas import tpu_sc as plsc

N_NODE, N_NET, E_PIN, E_NEAR = 10000, 4000, 20000, 100000
D_IN_NODE, D_IN_NET, D_IN_PIN = 128, 128, 16
H, HP, NT, NL = 64, 16, 8, 2

NODE_P, NET_P = 10240, 4096
EPP, ENP = 20480, 100352
NC, NS, L = 2, 16, 16
FW = 8
CN = 512
CP = 32
GW = (HP + 1) * H

_SC_PARAMS = pltpu.CompilerParams(
    use_tc_tiling_on_sc=False, needs_layout_passes=False)

_NEG = -1e30


def _mm(x, w, b, act, bm, out_dtype=jnp.float32):
    M, K = x.shape
    N = w.shape[1]

    def body(x_ref, w_ref, b_ref, o_ref):
        y = jnp.dot(x_ref[...], w_ref[...],
                    preferred_element_type=jnp.float32) + b_ref[...]
        o_ref[...] = act(y)

    return pl.pallas_call(
        body,
        grid=(M // bm,),
        in_specs=[
            pl.BlockSpec((bm, K), lambda i: (i, 0)),
            pl.BlockSpec((K, N), lambda i: (0, 0)),
            pl.BlockSpec((1, N), lambda i: (0, 0)),
        ],
        out_specs=pl.BlockSpec((bm, N), lambda i: (i, 0)),
        out_shape=jax.ShapeDtypeStruct((M, N), out_dtype),
    )(x, w, b.reshape(1, N))


def _lrelu(y):
    return jnp.where(y > 0, y, 0.01 * y)


def _hp_xs(node, dn3, wp, bp, bm=512):
    M = node.shape[0]

    def body(nd_ref, dn_ref, wp_ref, bp_ref, hp_ref, xs_ref):
        nd = nd_ref[...]
        hp_ref[...] = jnp.maximum(
            jnp.dot(nd, wp_ref[...], preferred_element_type=jnp.float32)
            + bp_ref[...], 0.0)
        d = dn_ref[0] + dn_ref[1]
        xs_ref[...] = nd * lax.rsqrt(jnp.maximum(d, 1.0))

    return pl.pallas_call(
        body,
        grid=(M // bm,),
        in_specs=[
            pl.BlockSpec((bm, H), lambda i: (i, 0)),
            pl.BlockSpec((2, bm, 1), lambda i: (0, i, 0)),
            pl.BlockSpec((H, H), lambda i: (0, 0)),
            pl.BlockSpec((1, H), lambda i: (0, 0)),
        ],
        out_specs=[
            pl.BlockSpec((bm, H), lambda i: (i, 0)),
            pl.BlockSpec((bm, H), lambda i: (i, 0)),
        ],
        out_shape=[
            jax.ShapeDtypeStruct((M, H), jnp.float32),
            jax.ShapeDtypeStruct((M, H), jnp.float32),
        ],
    )(node, dn3, wp, bp.reshape(1, H))


def _net_epilogue(agg_parts, dnt3, gc_w, gc_b, bm=512):
    M = agg_parts.shape[1]

    def body(a_ref, d_ref, w_ref, b_ref, o_ref):
        a = a_ref[0] + a_ref[1]
        d = d_ref[0] + d_ref[1]
        x = a * lax.rsqrt(jnp.maximum(d, 1.0))
        o_ref[...] = jnp.dot(
            x, w_ref[...], preferred_element_type=jnp.float32) + b_ref[...]

    return pl.pallas_call(
        body,
        grid=(M // bm,),
        in_specs=[
            pl.BlockSpec((2, bm, H), lambda i: (0, i, 0)),
            pl.BlockSpec((2, bm, 1), lambda i: (0, i, 0)),
            pl.BlockSpec((H, H), lambda i: (0, 0)),
            pl.BlockSpec((1, H), lambda i: (0, 0)),
        ],
        out_specs=pl.BlockSpec((bm, H), lambda i: (i, 0)),
        out_shape=jax.ShapeDtypeStruct((M, H), jnp.float32),
    )(agg_parts, dnt3, gc_w, gc_b.reshape(1, H))


def _node_epilogue(node, s_parts, m2, dn3, w_self, w_neigh, sage_b, nn_b,
                   bm=512):
    M = node.shape[0]

    def body(nd_ref, s_ref, m_ref, d_ref, ws_ref, wn_ref, sb_ref, nb_ref,
             o_ref):
        nd = nd_ref[...]
        s = s_ref[0] + s_ref[1]
        d = jnp.maximum(d_ref[0] + d_ref[1], 1.0)
        nn_out = s / d + nb_ref[...]
        m = jnp.maximum(jnp.maximum(m_ref[0], m_ref[1]),
                        jnp.maximum(m_ref[2], m_ref[3]))
        m = jnp.where(m > -1e29, m, 0.0)
        sage = (jnp.dot(nd, ws_ref[...], preferred_element_type=jnp.float32)
                + jnp.dot(m, wn_ref[...], preferred_element_type=jnp.float32)
                + sb_ref[...])
        o_ref[...] = jnp.maximum(nn_out, sage)

    return pl.pallas_call(
        body,
        grid=(M // bm,),
        in_specs=[
            pl.BlockSpec((bm, H), lambda i: (i, 0)),
            pl.BlockSpec((2, bm, H), lambda i: (0, i, 0)),
            pl.BlockSpec((4, bm, H), lambda i: (0, i, 0)),
            pl.BlockSpec((2, bm, 1), lambda i: (0, i, 0)),
            pl.BlockSpec((H, H), lambda i: (0, 0)),
            pl.BlockSpec((H, H), lambda i: (0, 0)),
            pl.BlockSpec((1, H), lambda i: (0, 0)),
            pl.BlockSpec((1, H), lambda i: (0, 0)),
        ],
        out_specs=pl.BlockSpec((bm, H), lambda i: (i, 0)),
        out_shape=jax.ShapeDtypeStruct((M, H), jnp.float32),
    )(node, s_parts, m2, dn3, w_self, w_neigh, sage_b.reshape(1, H),
      nn_b.reshape(1, H))


def _mlp(xn, node, o1a, o1b, o1_b, o2_w, o2_b, o3_w, o3_b, bm=512):
    M = xn.shape[0]

    def body(xn_ref, nd_ref, a_ref, b_ref, b1_ref, w2_ref, b2_ref, w3_ref,
             b3_ref, o_ref):
        h = jnp.tanh(
            jnp.dot(xn_ref[...], a_ref[...], preferred_element_type=jnp.float32)
            + jnp.dot(nd_ref[...], b_ref[...],
                      preferred_element_type=jnp.float32)
            + b1_ref[...])
        h = jnp.tanh(
            jnp.dot(h, w2_ref[...], preferred_element_type=jnp.float32)
            + b2_ref[...])
        y = (jnp.dot(h, w3_ref[...], preferred_element_type=jnp.float32)
             + b3_ref[...])
        o_ref[...] = jax.nn.sigmoid(y)

    return pl.pallas_call(
        body,
        grid=(M // bm,),
        in_specs=[
            pl.BlockSpec((bm, D_IN_NODE), lambda i: (i, 0)),
            pl.BlockSpec((bm, H), lambda i: (i, 0)),
            pl.BlockSpec((D_IN_NODE, H), lambda i: (0, 0)),
            pl.BlockSpec((H, H), lambda i: (0, 0)),
            pl.BlockSpec((1, H), lambda i: (0, 0)),
            pl.BlockSpec((H, H), lambda i: (0, 0)),
            pl.BlockSpec((1, H), lambda i: (0, 0)),
            pl.BlockSpec((H, NT), lambda i: (0, 0)),
            pl.BlockSpec((1, NT), lambda i: (0, 0)),
        ],
        out_specs=pl.BlockSpec((bm, NT), lambda i: (i, 0)),
        out_shape=jax.ShapeDtypeStruct((M, NT), jnp.float32),
    )(xn, node, o1a, o1b, o1_b.reshape(1, H), o2_w, o2_b.reshape(1, H),
      o3_w, o3_b.reshape(1, NT))


def _sc_mesh():
    return plsc.VectorSubcoreMesh(core_axis_name="c", subcore_axis_name="s")


def _deg_pass(src, dst):
    ept = EPP // (NC * NS)
    rpn = NODE_P // NS
    rpt = NET_P // NS

    @functools.partial(
        pl.kernel, mesh=_sc_mesh(), compiler_params=_SC_PARAMS,
        out_type=(jax.ShapeDtypeStruct((NC, NODE_P), jnp.float32),
                  jax.ShapeDtypeStruct((NC, NET_P), jnp.float32)),
        scratch_types=[
            pltpu.VMEM((ept,), jnp.int32),
            pltpu.VMEM((ept,), jnp.int32),
            pltpu.VMEM((ept,), jnp.float32),
            pltpu.VMEM((rpn,), jnp.float32),
            pltpu.VMEM_SHARED((NODE_P,), jnp.float32),
            pltpu.VMEM_SHARED((NET_P,), jnp.float32),
        ],
    )
    def k(src_hbm, dst_hbm, dn_hbm, dt_hbm, sidx, didx, ones, zb, accn, acct):
        c = lax.axis_index("c")
        s = lax.axis_index("s")
        zero = jnp.zeros((L,), jnp.float32)

        def zb_body(i, _):
            zb[pl.ds(i * L, L)] = zero
            return 0
        lax.fori_loop(0, rpn // L, zb_body, 0)
        pltpu.sync_copy(zb.at[pl.ds(0, rpn)], accn.at[pl.ds(s * rpn, rpn)])
        pltpu.sync_copy(zb.at[pl.ds(0, rpt)], acct.at[pl.ds(s * rpt, rpt)])
        plsc.subcore_barrier()

        one = jnp.ones((L,), jnp.float32)

        def ones_body(i, _):
            ones[pl.ds(i * L, L)] = one
            return 0
        lax.fori_loop(0, ept // L, ones_body, 0)

        base = (c * NS + s) * ept
        pltpu.sync_copy(src_hbm.at[pl.ds(base, ept)], sidx)
        pltpu.sync_copy(dst_hbm.at[pl.ds(base, ept)], didx)
        pltpu.sync_copy(ones, accn.at[sidx], add=True)
        pltpu.sync_copy(ones, acct.at[didx], add=True)
        plsc.subcore_barrier()
        pltpu.sync_copy(accn.at[pl.ds(s * rpn, rpn)],
                        dn_hbm.at[c, pl.ds(s * rpn, rpn)])
        pltpu.sync_copy(acct.at[pl.ds(s * rpt, rpt)],
                        dt_hbm.at[c, pl.ds(s * rpt, rpt)])

    return k(src, dst)


def _agg_pass(src, dst, xs, zeros2d):
    ept = EPP // (NC * NS)
    rpt = NET_P // NS

    @functools.partial(
        pl.kernel, mesh=_sc_mesh(), compiler_params=_SC_PARAMS,
        out_type=jax.ShapeDtypeStruct((NC, NET_P, H), jnp.float32),
        scratch_types=[
            pltpu.VMEM((ept,), jnp.int32),
            pltpu.VMEM((ept,), jnp.int32),
            pltpu.VMEM((ept, H), jnp.float32),
            pltpu.VMEM_SHARED((NET_P, H), jnp.float32),
            pltpu.SemaphoreType.DMA,
        ],
    )
    def k(src_hbm, dst_hbm, xs_hbm, z_hbm, agg_out, sidx, didx, buf, agg_sh,
          sem0):
        c = lax.axis_index("c")
        s = lax.axis_index("s")
        pltpu.sync_copy(z_hbm.at[pl.ds(s * rpt, rpt)],
                        agg_sh.at[pl.ds(s * rpt, rpt)])
        plsc.subcore_barrier()
        tile_base = (c * NS + s) * ept
        pltpu.sync_copy(src_hbm.at[pl.ds(tile_base, ept)], sidx)
        pltpu.sync_copy(dst_hbm.at[pl.ds(tile_base, ept)], didx)
        pltpu.async_copy(xs_hbm.at[sidx], buf, sem0).wait()
        pltpu.sync_copy(buf, agg_sh.at[didx], add=True)
        plsc.subcore_barrier()
        pltpu.sync_copy(agg_sh.at[pl.ds(s * rpt, rpt)],
                        agg_out.at[c, pl.ds(s * rpt, rpt)])

    return k(src, dst, xs, zeros2d)


def _nnconv_pass(src, dst, gaug, pinw, zeros2d):
    ept = EPP // (NC * NS)
    GC = 32
    npair = ept // (2 * GC)
    rpn = NODE_P // NS

    @functools.partial(
        pl.kernel, mesh=_sc_mesh(), compiler_params=_SC_PARAMS,
        out_type=jax.ShapeDtypeStruct((NC, NODE_P, H), jnp.float32),
        scratch_types=[
            pltpu.VMEM((ept,), jnp.int32),
            pltpu.VMEM((ept,), jnp.int32),
            pltpu.VMEM((ept, HP), jnp.float32),
            pltpu.VMEM((ept, H), jnp.float32),
            pltpu.VMEM((GC, GW), jnp.bfloat16),
            pltpu.VMEM((GC, GW), jnp.bfloat16),
            pltpu.VMEM_SHARED((NODE_P, H), jnp.float32),
            pltpu.SemaphoreType.DMA,
            pltpu.SemaphoreType.DMA,
        ],
    )
    def k(src_hbm, dst_hbm, g_hbm, pin_hbm, z_hbm, s_out,
          sidx, didx, pinb, buf, gr0, gr1, s_sh, sem0, sem1):
        c = lax.axis_index("c")
        s = lax.axis_index("s")
        iota = lax.iota(jnp.int32, L)
        pltpu.sync_copy(z_hbm.at[pl.ds(s * rpn, rpn)],
                        s_sh.at[pl.ds(s * rpn, rpn)])
        plsc.subcore_barrier()

        tile_base = (c * NS + s) * ept
        pltpu.sync_copy(src_hbm.at[pl.ds(tile_base, ept)], sidx)
        pltpu.sync_copy(dst_hbm.at[pl.ds(tile_base, ept)], didx)
        pltpu.sync_copy(pin_hbm.at[pl.ds(tile_base, ept)], pinb)

        def gidx_ref(t):
            return didx.at[pl.ds(t * GC, GC)]

        def edge_body(grbuf, ebase, e, _):
            eg = ebase + e
            pw = pinb[eg, pl.ds(0, HP)]
            accs = [None] * (H // L)
            for c2 in range(H // (2 * L)):
                gb = grbuf[e, pl.ds(HP * H + c2 * 2 * L, 2 * L)]
                a, b = plsc.unpack(gb, format=plsc.PackFormat.INTERLEAVED)
                accs[2 * c2] = a
                accs[2 * c2 + 1] = b
            for kk in range(HP):
                w = pw[iota * 0 + kk]
                for c2 in range(H // (2 * L)):
                    gb = grbuf[e, pl.ds(kk * H + c2 * 2 * L, 2 * L)]
                    a, b = plsc.unpack(gb, format=plsc.PackFormat.INTERLEAVED)
                    accs[2 * c2] = accs[2 * c2] + w * a
                    accs[2 * c2 + 1] = accs[2 * c2 + 1] + w * b
            for c4 in range(H // L):
                buf[eg, pl.ds(c4 * L, L)] = accs[c4]
            return 0

        pltpu.async_copy(g_hbm.at[gidx_ref(0)], gr0, sem0)

        def round_body(q, _):
            t0 = 2 * q
            pltpu.async_copy(g_hbm.at[gidx_ref(t0 + 1)], gr1, sem1)
            pltpu.make_async_copy(g_hbm.at[gidx_ref(t0)], gr0, sem0).wait()
            lax.fori_loop(0, GC, functools.partial(edge_body, gr0, t0 * GC), 0)

            @pl.when(q < npair - 1)
            def _():
                pltpu.async_copy(g_hbm.at[gidx_ref(t0 + 2)], gr0, sem0)
            pltpu.make_async_copy(g_hbm.at[gidx_ref(t0 + 1)], gr1, sem1).wait()
            lax.fori_loop(0, GC, functools.partial(edge_body, gr1,
                                                   (t0 + 1) * GC), 0)
            return 0
        lax.fori_loop(0, npair, round_body, 0)
        pltpu.sync_copy(buf, s_sh.at[sidx], add=True)
        plsc.subcore_barrier()
        pltpu.sync_copy(s_sh.at[pl.ds(s * rpn, rpn)],
                        s_out.at[c, pl.ds(s * rpn, rpn)])

    return k(src, dst, gaug, pinw, zeros2d)


def _near_pass(hpt, src, dst):
    ept = ENP // 4
    SUP = ept // 2
    CN2 = 224
    nch = SUP // CN2
    AW = NODE_P * FW

    @functools.partial(
        pl.kernel, mesh=_sc_mesh(), compiler_params=_SC_PARAMS,
        out_type=jax.ShapeDtypeStruct((NC, 2, 8, AW), jnp.float32),
        scratch_types=[
            pltpu.VMEM((SUP,), jnp.int32),
            pltpu.VMEM((SUP,), jnp.int32),
            pltpu.VMEM((CN2, FW), jnp.float32),
            pltpu.VMEM((CN2, FW), jnp.float32),
            pltpu.VMEM((AW,), jnp.float32),
            pltpu.SemaphoreType.DMA,
            pltpu.SemaphoreType.DMA,
        ],
    )
    def k(hpt_hbm, src_hbm, dst_hbm, out_hbm, sidx, didx, rows0, rows1, acc,
          sem0, sem1):
        c = lax.axis_index("c")
        s = lax.axis_index("s")
        fc = s % 8
        es = s // 8
        ebase = (c * 2 + es) * ept
        iota = lax.iota(jnp.int32, L)
        half = iota // FW
        lane8 = iota % FW
        swap8 = iota ^ FW

        neg = jnp.full((L,), _NEG, jnp.float32)

        def initbody(i, _):
            acc[pl.ds(i * L, L)] = neg
            return 0
        lax.fori_loop(0, AW // L, initbody, 0)

        def gidx_ref(t):
            return sidx.at[pl.ds(t * CN2, CN2)]

        def pair8(rbuf, cbase, i, _):
            dblk = didx[pl.ds(cbase + i * L, L)]
            for u in range(8):
                d1 = dblk[2 * u + half]
                d2 = dblk[2 * u + (1 - half)]
                rr = i * L + 2 * u + half
                hp2 = plsc.load_gather(rbuf, [rr, lane8])
                hps = hp2[swap8]
                val = jnp.where(d1 == d2, jnp.maximum(hp2, hps), hp2)
                ia = d1 * FW + lane8
                cur = plsc.load_gather(acc, [ia])
                plsc.store_scatter(acc, [ia], jnp.maximum(cur, val))
            return 0

        for sup in range(2):
            base = ebase + sup * SUP
            pltpu.sync_copy(src_hbm.at[pl.ds(base, SUP)], sidx)
            pltpu.sync_copy(dst_hbm.at[pl.ds(base, SUP)], didx)

            def shiftbody(i, _):
                sidx[pl.ds(i * L, L)] = sidx[pl.ds(i * L, L)] + fc * NODE_P
                return 0
            lax.fori_loop(0, SUP // L, shiftbody, 0)

            pltpu.async_copy(hpt_hbm.at[gidx_ref(0)], rows0, sem0)

            def round_body(q, _):
                t0 = 2 * q
                pltpu.async_copy(hpt_hbm.at[gidx_ref(t0 + 1)], rows1, sem1)
                pltpu.make_async_copy(
                    hpt_hbm.at[gidx_ref(t0)], rows0, sem0).wait()
                lax.fori_loop(0, CN2 // L,
                              functools.partial(pair8, rows0, t0 * CN2), 0)

                @pl.when(q < nch // 2 - 1)
                def _():
                    pltpu.async_copy(hpt_hbm.at[gidx_ref(t0 + 2)], rows0, sem0)
                pltpu.make_async_copy(
                    hpt_hbm.at[gidx_ref(t0 + 1)], rows1, sem1).wait()
                lax.fori_loop(0, CN2 // L,
                              functools.partial(pair8, rows1, (t0 + 1) * CN2),
                              0)
                return 0
            lax.fori_loop(0, nch // 2, round_body, 0)

        pltpu.sync_copy(acc, out_hbm.at[c, es, fc])

    return k(hpt, src, dst)


def _pad_rows(x, rows):
    return jnp.pad(x, ((0, rows - x.shape[0]), (0, 0)))


def kernel(in_node_feat, in_net_feat, in_pin_feat, pins_src, pins_dst,
           near_src, near_dst, params):
    p = params
    f32 = jnp.float32

    in_node_p = _pad_rows(in_node_feat.astype(f32), NODE_P)
    in_net_p = _pad_rows(in_net_feat.astype(f32), NET_P)
    in_pin_p = _pad_rows(in_pin_feat.astype(f32), EPP)

    i32 = jnp.int32
    psrc = jnp.concatenate([pins_src.astype(i32),
                            jnp.full((EPP - E_PIN,), N_NODE, i32)])
    pdst = jnp.concatenate([pins_dst.astype(i32),
                            jnp.full((EPP - E_PIN,), N_NET, i32)])
    nsrc = jnp.concatenate([near_src.astype(i32),
                            jnp.zeros((ENP - E_NEAR,), i32)])
    ndst = jnp.concatenate([near_dst.astype(i32),
                            jnp.full((ENP - E_NEAR,), N_NODE, i32)])
    zeros2d = jnp.zeros((NODE_P, H), f32)

    node = _mm(in_node_p, p['node_W'], p['node_b'], _lrelu, 512)
    net = _mm(in_net_p, p['net_W'], p['net_b'], _lrelu, 512)
    pinw = _mm(in_pin_p, p['pin_W'], p['pin_b'], _lrelu, 2048)

    dn_parts, dnt_parts = _deg_pass(psrc, pdst)
    dn3 = dn_parts.reshape(NC, NODE_P, 1)
    dnt3 = dnt_parts.reshape(NC, NET_P, 1)

    for l in range(NL):
        lp = p['layers'][l]
        t = lp['lin2_W'].reshape(HP, H, H)
        waug = jnp.concatenate(
            [t.transpose(1, 0, 2).reshape(H, HP * H),
             lp['lin2_b'].reshape(H, H)], axis=1)
        waug = waug.reshape(H, GW // 32, 2, 16).transpose(0, 1, 3, 2).reshape(
            H, GW)

        hp, xs = _hp_xs(node, dn3, lp['sage_Wp'], lp['sage_bp'])
        hpt = hp.reshape(NODE_P, 8, FW).transpose(1, 0, 2).reshape(
            8 * NODE_P, FW)
        gaug = _mm(net, waug, jnp.zeros((GW,), f32),
                   lambda y: y.astype(jnp.bfloat16), 512,
                   out_dtype=jnp.bfloat16)

        agg_parts = _agg_pass(psrc, pdst, xs, zeros2d)
        s_parts = _nnconv_pass(psrc, pdst, gaug, pinw, zeros2d)
        m_parts = _near_pass(hpt, nsrc, ndst)
        m2 = m_parts.reshape(NC * 2, 8, NODE_P, FW).transpose(
            0, 2, 1, 3).reshape(NC * 2, NODE_P, H)

        net = _net_epilogue(agg_parts, dnt3, lp['gc_W'], lp['gc_b'])
        node = _node_epilogue(node, s_parts, m2, dn3, lp['sage_Wself'],
                              lp['sage_Wneigh'], lp['sage_b'], lp['nn_b'])

    o1a = p['o1_W'][:D_IN_NODE]
    o1b = p['o1_W'][D_IN_NODE:]
    out = _mlp(in_node_p, node, o1a, o1b, p['o1_b'], p['o2_W'], p['o2_b'],
               p['o3_W'], p['o3_b'])
    return out[:N_NODE]


# speedup vs baseline: 1.1226x; 1.1226x over previous
import functools

import jax
import jax.numpy as jnp
from jax import lax
from jax.experimental import pallas as pl
from jax.experimental.pallas import tpu as pltpu
from jax.experimental.pallas import tpu_sc as plsc

N_NODE, N_NET, E_PIN, E_NEAR = 10000, 4000, 20000, 100000
D_IN_NODE, D_IN_NET, D_IN_PIN = 128, 128, 16
H, HP, NT, NL = 64, 16, 8, 2

NODE_P, NET_P = 10240, 4096
EPP, ENP = 20480, 100352
NC, NS, L = 2, 16, 16
FW = 8
CN = 512
CP = 32
GW = (HP + 1) * H

_SC_PARAMS = pltpu.CompilerParams(
    use_tc_tiling_on_sc=False, needs_layout_passes=False)

_NEG = -1e30


def _mm(x, w, b, act, bm, out_dtype=jnp.float32):
    M, K = x.shape
    N = w.shape[1]

    def body(x_ref, w_ref, b_ref, o_ref):
        y = jnp.dot(x_ref[...], w_ref[...],
                    preferred_element_type=jnp.float32) + b_ref[...]
        o_ref[...] = act(y)

    return pl.pallas_call(
        body,
        grid=(M // bm,),
        in_specs=[
            pl.BlockSpec((bm, K), lambda i: (i, 0)),
            pl.BlockSpec((K, N), lambda i: (0, 0)),
            pl.BlockSpec((1, N), lambda i: (0, 0)),
        ],
        out_specs=pl.BlockSpec((bm, N), lambda i: (i, 0)),
        out_shape=jax.ShapeDtypeStruct((M, N), out_dtype),
    )(x, w, b.reshape(1, N))


def _lrelu(y):
    return jnp.where(y > 0, y, 0.01 * y)


def _hp_xs(node, dn3, wp, bp, bm=512):
    M = node.shape[0]

    def body(nd_ref, dn_ref, wp_ref, bp_ref, hp_ref, xs_ref):
        nd = nd_ref[...]
        hp_ref[...] = jnp.maximum(
            jnp.dot(nd, wp_ref[...], preferred_element_type=jnp.float32)
            + bp_ref[...], 0.0)
        d = dn_ref[0] + dn_ref[1]
        xs_ref[...] = nd * lax.rsqrt(jnp.maximum(d, 1.0))

    return pl.pallas_call(
        body,
        grid=(M // bm,),
        in_specs=[
            pl.BlockSpec((bm, H), lambda i: (i, 0)),
            pl.BlockSpec((2, bm, 1), lambda i: (0, i, 0)),
            pl.BlockSpec((H, H), lambda i: (0, 0)),
            pl.BlockSpec((1, H), lambda i: (0, 0)),
        ],
        out_specs=[
            pl.BlockSpec((bm, H), lambda i: (i, 0)),
            pl.BlockSpec((bm, H), lambda i: (i, 0)),
        ],
        out_shape=[
            jax.ShapeDtypeStruct((M, H), jnp.float32),
            jax.ShapeDtypeStruct((M, H), jnp.float32),
        ],
    )(node, dn3, wp, bp.reshape(1, H))


def _net_epilogue(agg_parts, dnt3, gc_w, gc_b, bm=512):
    M = agg_parts.shape[1]

    def body(a_ref, d_ref, w_ref, b_ref, o_ref):
        a = a_ref[0] + a_ref[1]
        d = d_ref[0] + d_ref[1]
        x = a * lax.rsqrt(jnp.maximum(d, 1.0))
        o_ref[...] = jnp.dot(
            x, w_ref[...], preferred_element_type=jnp.float32) + b_ref[...]

    return pl.pallas_call(
        body,
        grid=(M // bm,),
        in_specs=[
            pl.BlockSpec((2, bm, H), lambda i: (0, i, 0)),
            pl.BlockSpec((2, bm, 1), lambda i: (0, i, 0)),
            pl.BlockSpec((H, H), lambda i: (0, 0)),
            pl.BlockSpec((1, H), lambda i: (0, 0)),
        ],
        out_specs=pl.BlockSpec((bm, H), lambda i: (i, 0)),
        out_shape=jax.ShapeDtypeStruct((M, H), jnp.float32),
    )(agg_parts, dnt3, gc_w, gc_b.reshape(1, H))


def _node_epilogue(node, s_parts, m2, dn3, w_self, w_neigh, sage_b, nn_b,
                   bm=512):
    M = node.shape[0]

    def body(nd_ref, s_ref, m_ref, d_ref, ws_ref, wn_ref, sb_ref, nb_ref,
             o_ref):
        nd = nd_ref[...]
        s = s_ref[0] + s_ref[1]
        d = jnp.maximum(d_ref[0] + d_ref[1], 1.0)
        nn_out = s / d + nb_ref[...]
        m = jnp.maximum(jnp.maximum(m_ref[0], m_ref[1]),
                        jnp.maximum(m_ref[2], m_ref[3]))
        m = jnp.where(m > -1e29, m, 0.0)
        sage = (jnp.dot(nd, ws_ref[...], preferred_element_type=jnp.float32)
                + jnp.dot(m, wn_ref[...], preferred_element_type=jnp.float32)
                + sb_ref[...])
        o_ref[...] = jnp.maximum(nn_out, sage)

    return pl.pallas_call(
        body,
        grid=(M // bm,),
        in_specs=[
            pl.BlockSpec((bm, H), lambda i: (i, 0)),
            pl.BlockSpec((2, bm, H), lambda i: (0, i, 0)),
            pl.BlockSpec((4, bm, H), lambda i: (0, i, 0)),
            pl.BlockSpec((2, bm, 1), lambda i: (0, i, 0)),
            pl.BlockSpec((H, H), lambda i: (0, 0)),
            pl.BlockSpec((H, H), lambda i: (0, 0)),
            pl.BlockSpec((1, H), lambda i: (0, 0)),
            pl.BlockSpec((1, H), lambda i: (0, 0)),
        ],
        out_specs=pl.BlockSpec((bm, H), lambda i: (i, 0)),
        out_shape=jax.ShapeDtypeStruct((M, H), jnp.float32),
    )(node, s_parts, m2, dn3, w_self, w_neigh, sage_b.reshape(1, H),
      nn_b.reshape(1, H))


def _mlp(xn, node, o1a, o1b, o1_b, o2_w, o2_b, o3_w, o3_b, bm=512):
    M = xn.shape[0]

    def body(xn_ref, nd_ref, a_ref, b_ref, b1_ref, w2_ref, b2_ref, w3_ref,
             b3_ref, o_ref):
        h = jnp.tanh(
            jnp.dot(xn_ref[...], a_ref[...], preferred_element_type=jnp.float32)
            + jnp.dot(nd_ref[...], b_ref[...],
                      preferred_element_type=jnp.float32)
            + b1_ref[...])
        h = jnp.tanh(
            jnp.dot(h, w2_ref[...], preferred_element_type=jnp.float32)
            + b2_ref[...])
        y = (jnp.dot(h, w3_ref[...], preferred_element_type=jnp.float32)
             + b3_ref[...])
        o_ref[...] = jax.nn.sigmoid(y)

    return pl.pallas_call(
        body,
        grid=(M // bm,),
        in_specs=[
            pl.BlockSpec((bm, D_IN_NODE), lambda i: (i, 0)),
            pl.BlockSpec((bm, H), lambda i: (i, 0)),
            pl.BlockSpec((D_IN_NODE, H), lambda i: (0, 0)),
            pl.BlockSpec((H, H), lambda i: (0, 0)),
            pl.BlockSpec((1, H), lambda i: (0, 0)),
            pl.BlockSpec((H, H), lambda i: (0, 0)),
            pl.BlockSpec((1, H), lambda i: (0, 0)),
            pl.BlockSpec((H, NT), lambda i: (0, 0)),
            pl.BlockSpec((1, NT), lambda i: (0, 0)),
        ],
        out_specs=pl.BlockSpec((bm, NT), lambda i: (i, 0)),
        out_shape=jax.ShapeDtypeStruct((M, NT), jnp.float32),
    )(xn, node, o1a, o1b, o1_b.reshape(1, H), o2_w, o2_b.reshape(1, H),
      o3_w, o3_b.reshape(1, NT))


def _sc_mesh():
    return plsc.VectorSubcoreMesh(core_axis_name="c", subcore_axis_name="s")


def _deg_pass(src, dst):
    ept = EPP // (NC * NS)
    rpn = NODE_P // NS
    rpt = NET_P // NS

    @functools.partial(
        pl.kernel, mesh=_sc_mesh(), compiler_params=_SC_PARAMS,
        out_type=(jax.ShapeDtypeStruct((NC, NODE_P), jnp.float32),
                  jax.ShapeDtypeStruct((NC, NET_P), jnp.float32)),
        scratch_types=[
            pltpu.VMEM((ept,), jnp.int32),
            pltpu.VMEM((ept,), jnp.int32),
            pltpu.VMEM((ept,), jnp.float32),
            pltpu.VMEM((rpn,), jnp.float32),
            pltpu.VMEM_SHARED((NODE_P,), jnp.float32),
            pltpu.VMEM_SHARED((NET_P,), jnp.float32),
        ],
    )
    def k(src_hbm, dst_hbm, dn_hbm, dt_hbm, sidx, didx, ones, zb, accn, acct):
        c = lax.axis_index("c")
        s = lax.axis_index("s")
        zero = jnp.zeros((L,), jnp.float32)

        def zb_body(i, _):
            zb[pl.ds(i * L, L)] = zero
            return 0
        lax.fori_loop(0, rpn // L, zb_body, 0)
        pltpu.sync_copy(zb.at[pl.ds(0, rpn)], accn.at[pl.ds(s * rpn, rpn)])
        pltpu.sync_copy(zb.at[pl.ds(0, rpt)], acct.at[pl.ds(s * rpt, rpt)])
        plsc.subcore_barrier()

        one = jnp.ones((L,), jnp.float32)

        def ones_body(i, _):
            ones[pl.ds(i * L, L)] = one
            return 0
        lax.fori_loop(0, ept // L, ones_body, 0)

        base = (c * NS + s) * ept
        pltpu.sync_copy(src_hbm.at[pl.ds(base, ept)], sidx)
        pltpu.sync_copy(dst_hbm.at[pl.ds(base, ept)], didx)
        pltpu.sync_copy(ones, accn.at[sidx], add=True)
        pltpu.sync_copy(ones, acct.at[didx], add=True)
        plsc.subcore_barrier()
        pltpu.sync_copy(accn.at[pl.ds(s * rpn, rpn)],
                        dn_hbm.at[c, pl.ds(s * rpn, rpn)])
        pltpu.sync_copy(acct.at[pl.ds(s * rpt, rpt)],
                        dt_hbm.at[c, pl.ds(s * rpt, rpt)])

    return k(src, dst)


def _agg_pass(src, dst, xs, zeros2d):
    ept = EPP // (NC * NS)
    rpt = NET_P // NS

    @functools.partial(
        pl.kernel, mesh=_sc_mesh(), compiler_params=_SC_PARAMS,
        out_type=jax.ShapeDtypeStruct((NC, NET_P, H), jnp.float32),
        scratch_types=[
            pltpu.VMEM((ept,), jnp.int32),
            pltpu.VMEM((ept,), jnp.int32),
            pltpu.VMEM((ept, H), jnp.float32),
            pltpu.VMEM_SHARED((NET_P, H), jnp.float32),
            pltpu.SemaphoreType.DMA,
        ],
    )
    def k(src_hbm, dst_hbm, xs_hbm, z_hbm, agg_out, sidx, didx, buf, agg_sh,
          sem0):
        c = lax.axis_index("c")
        s = lax.axis_index("s")
        pltpu.sync_copy(z_hbm.at[pl.ds(s * rpt, rpt)],
                        agg_sh.at[pl.ds(s * rpt, rpt)])
        plsc.subcore_barrier()
        tile_base = (c * NS + s) * ept
        pltpu.sync_copy(src_hbm.at[pl.ds(tile_base, ept)], sidx)
        pltpu.sync_copy(dst_hbm.at[pl.ds(tile_base, ept)], didx)
        pltpu.async_copy(xs_hbm.at[sidx], buf, sem0).wait()
        pltpu.sync_copy(buf, agg_sh.at[didx], add=True)
        plsc.subcore_barrier()
        pltpu.sync_copy(agg_sh.at[pl.ds(s * rpt, rpt)],
                        agg_out.at[c, pl.ds(s * rpt, rpt)])

    return k(src, dst, xs, zeros2d)


def _nnconv_pass(src, dst, gaug, pinw, zeros2d):
    ept = EPP // (NC * NS)
    GC = 16
    npair = ept // (2 * GC)
    rpn = NODE_P // NS

    @functools.partial(
        pl.kernel, mesh=_sc_mesh(), compiler_params=_SC_PARAMS,
        out_type=jax.ShapeDtypeStruct((NC, NODE_P, H), jnp.float32),
        scratch_types=[
            pltpu.VMEM((ept,), jnp.int32),
            pltpu.VMEM((ept,), jnp.int32),
            pltpu.VMEM((ept, HP), jnp.float32),
            pltpu.VMEM((ept, H), jnp.float32),
            pltpu.VMEM((GC, GW), jnp.float32),
            pltpu.VMEM((GC, GW), jnp.float32),
            pltpu.VMEM_SHARED((NODE_P, H), jnp.float32),
            pltpu.SemaphoreType.DMA,
            pltpu.SemaphoreType.DMA,
        ],
    )
    def k(src_hbm, dst_hbm, g_hbm, pin_hbm, z_hbm, s_out,
          sidx, didx, pinb, buf, gr0, gr1, s_sh, sem0, sem1):
        c = lax.axis_index("c")
        s = lax.axis_index("s")
        iota = lax.iota(jnp.int32, L)
        pltpu.sync_copy(z_hbm.at[pl.ds(s * rpn, rpn)],
                        s_sh.at[pl.ds(s * rpn, rpn)])
        plsc.subcore_barrier()

        tile_base = (c * NS + s) * ept
        pltpu.sync_copy(src_hbm.at[pl.ds(tile_base, ept)], sidx)
        pltpu.sync_copy(dst_hbm.at[pl.ds(tile_base, ept)], didx)
        pltpu.sync_copy(pin_hbm.at[pl.ds(tile_base, ept)], pinb)

        def gidx_ref(t):
            return didx.at[pl.ds(t * GC, GC)]

        def edge_body(grbuf, ebase, e, _):
            eg = ebase + e
            pw = pinb[eg, pl.ds(0, HP)]
            accs = [grbuf[e, pl.ds(HP * H + c4 * L, L)]
                    for c4 in range(H // L)]
            for kk in range(HP):
                w = pw[iota * 0 + kk]
                for c4 in range(H // L):
                    accs[c4] = accs[c4] + w * grbuf[
                        e, pl.ds(kk * H + c4 * L, L)]
            for c4 in range(H // L):
                buf[eg, pl.ds(c4 * L, L)] = accs[c4]
            return 0

        pltpu.async_copy(g_hbm.at[gidx_ref(0)], gr0, sem0)

        def round_body(q, _):
            t0 = 2 * q
            pltpu.async_copy(g_hbm.at[gidx_ref(t0 + 1)], gr1, sem1)
            pltpu.make_async_copy(g_hbm.at[gidx_ref(t0)], gr0, sem0).wait()
            lax.fori_loop(0, GC, functools.partial(edge_body, gr0, t0 * GC), 0)

            @pl.when(q < npair - 1)
            def _():
                pltpu.async_copy(g_hbm.at[gidx_ref(t0 + 2)], gr0, sem0)
            pltpu.make_async_copy(g_hbm.at[gidx_ref(t0 + 1)], gr1, sem1).wait()
            lax.fori_loop(0, GC, functools.partial(edge_body, gr1,
                                                   (t0 + 1) * GC), 0)
            return 0
        lax.fori_loop(0, npair, round_body, 0)
        pltpu.sync_copy(buf, s_sh.at[sidx], add=True)
        plsc.subcore_barrier()
        pltpu.sync_copy(s_sh.at[pl.ds(s * rpn, rpn)],
                        s_out.at[c, pl.ds(s * rpn, rpn)])

    return k(src, dst, gaug, pinw, zeros2d)


def _near_pass(hpt, src, dst):
    ept = ENP // 4
    SUP = ept // 2
    CN2 = 224
    nch = SUP // CN2
    AW = NODE_P * FW

    @functools.partial(
        pl.kernel, mesh=_sc_mesh(), compiler_params=_SC_PARAMS,
        out_type=jax.ShapeDtypeStruct((NC, 2, 8, AW), jnp.float32),
        scratch_types=[
            pltpu.VMEM((SUP,), jnp.int32),
            pltpu.VMEM((SUP,), jnp.int32),
            pltpu.VMEM((CN2, FW), jnp.float32),
            pltpu.VMEM((CN2, FW), jnp.float32),
            pltpu.VMEM((AW,), jnp.float32),
            pltpu.SemaphoreType.DMA,
            pltpu.SemaphoreType.DMA,
        ],
    )
    def k(hpt_hbm, src_hbm, dst_hbm, out_hbm, sidx, didx, rows0, rows1, acc,
          sem0, sem1):
        c = lax.axis_index("c")
        s = lax.axis_index("s")
        fc = s % 8
        es = s // 8
        ebase = (c * 2 + es) * ept
        iota = lax.iota(jnp.int32, L)
        half = iota // FW
        lane8 = iota % FW
        swap8 = iota ^ FW

        neg = jnp.full((L,), _NEG, jnp.float32)

        def initbody(i, _):
            acc[pl.ds(i * L, L)] = neg
            return 0
        lax.fori_loop(0, AW // L, initbody, 0)

        def gidx_ref(t):
            return sidx.at[pl.ds(t * CN2, CN2)]

        def pair8(rbuf, cbase, i, _):
            dblk = didx[pl.ds(cbase + i * L, L)]
            vals = []
            ias = []
            for u in range(8):
                d1 = dblk[2 * u + half]
                d2 = dblk[2 * u + (1 - half)]
                rr = i * L + 2 * u + half
                hp2 = plsc.load_gather(rbuf, [rr, lane8])
                hps = hp2[swap8]
                vals.append(jnp.where(d1 == d2, jnp.maximum(hp2, hps), hp2))
                ias.append(d1 * FW + lane8)
            for u in range(8):
                cur = plsc.load_gather(acc, [ias[u]])
                plsc.store_scatter(acc, [ias[u]], jnp.maximum(cur, vals[u]))
            return 0

        for sup in range(2):
            base = ebase + sup * SUP
            pltpu.sync_copy(src_hbm.at[pl.ds(base, SUP)], sidx)
            pltpu.sync_copy(dst_hbm.at[pl.ds(base, SUP)], didx)

            def shiftbody(i, _):
                sidx[pl.ds(i * L, L)] = sidx[pl.ds(i * L, L)] + fc * NODE_P
                return 0
            lax.fori_loop(0, SUP // L, shiftbody, 0)

            pltpu.async_copy(hpt_hbm.at[gidx_ref(0)], rows0, sem0)

            def round_body(q, _):
                t0 = 2 * q
                pltpu.async_copy(hpt_hbm.at[gidx_ref(t0 + 1)], rows1, sem1)
                pltpu.make_async_copy(
                    hpt_hbm.at[gidx_ref(t0)], rows0, sem0).wait()
                lax.fori_loop(0, CN2 // L,
                              functools.partial(pair8, rows0, t0 * CN2), 0)

                @pl.when(q < nch // 2 - 1)
                def _():
                    pltpu.async_copy(hpt_hbm.at[gidx_ref(t0 + 2)], rows0, sem0)
                pltpu.make_async_copy(
                    hpt_hbm.at[gidx_ref(t0 + 1)], rows1, sem1).wait()
                lax.fori_loop(0, CN2 // L,
                              functools.partial(pair8, rows1, (t0 + 1) * CN2),
                              0)
                return 0
            lax.fori_loop(0, nch // 2, round_body, 0)

        pltpu.sync_copy(acc, out_hbm.at[c, es, fc])

    return k(hpt, src, dst)


def _pad_rows(x, rows):
    return jnp.pad(x, ((0, rows - x.shape[0]), (0, 0)))


def kernel(in_node_feat, in_net_feat, in_pin_feat, pins_src, pins_dst,
           near_src, near_dst, params):
    p = params
    f32 = jnp.float32

    in_node_p = _pad_rows(in_node_feat.astype(f32), NODE_P)
    in_net_p = _pad_rows(in_net_feat.astype(f32), NET_P)
    in_pin_p = _pad_rows(in_pin_feat.astype(f32), EPP)

    i32 = jnp.int32
    psrc = jnp.concatenate([pins_src.astype(i32),
                            jnp.full((EPP - E_PIN,), N_NODE, i32)])
    pdst = jnp.concatenate([pins_dst.astype(i32),
                            jnp.full((EPP - E_PIN,), N_NET, i32)])
    nsrc = jnp.concatenate([near_src.astype(i32),
                            jnp.zeros((ENP - E_NEAR,), i32)])
    ndst = jnp.concatenate([near_dst.astype(i32),
                            jnp.full((ENP - E_NEAR,), N_NODE, i32)])
    zeros2d = jnp.zeros((NODE_P, H), f32)

    node = _mm(in_node_p, p['node_W'], p['node_b'], _lrelu, 512)
    net = _mm(in_net_p, p['net_W'], p['net_b'], _lrelu, 512)
    pinw = _mm(in_pin_p, p['pin_W'], p['pin_b'], _lrelu, 2048)

    dn_parts, dnt_parts = _deg_pass(psrc, pdst)
    dn3 = dn_parts.reshape(NC, NODE_P, 1)
    dnt3 = dnt_parts.reshape(NC, NET_P, 1)

    for l in range(NL):
        lp = p['layers'][l]
        t = lp['lin2_W'].reshape(HP, H, H)
        waug = jnp.concatenate(
            [t.transpose(1, 0, 2).reshape(H, HP * H),
             lp['lin2_b'].reshape(H, H)], axis=1)

        hp, xs = _hp_xs(node, dn3, lp['sage_Wp'], lp['sage_bp'])
        hpt = hp.reshape(NODE_P, 8, FW).transpose(1, 0, 2).reshape(
            8 * NODE_P, FW)
        gaug = _mm(net, waug, jnp.zeros((GW,), f32), lambda y: y, 512)

        agg_parts = _agg_pass(psrc, pdst, xs, zeros2d)
        s_parts = _nnconv_pass(psrc, pdst, gaug, pinw, zeros2d)
        m_parts = _near_pass(hpt, nsrc, ndst)
        m2 = m_parts.reshape(NC * 2, 8, NODE_P, FW).transpose(
            0, 2, 1, 3).reshape(NC * 2, NODE_P, H)

        net = _net_epilogue(agg_parts, dnt3, lp['gc_W'], lp['gc_b'])
        node = _node_epilogue(node, s_parts, m2, dn3, lp['sage_Wself'],
                              lp['sage_Wneigh'], lp['sage_b'], lp['nn_b'])

    o1a = p['o1_W'][:D_IN_NODE]
    o1b = p['o1_W'][D_IN_NODE:]
    out = _mlp(in_node_p, node, o1a, o1b, p['o1_b'], p['o2_W'], p['o2_b'],
               p['o3_W'], p['o3_b'])
    return out[:N_NODE]


# speedup vs baseline: 1.1325x; 1.0088x over previous
import functools

import jax
import jax.numpy as jnp
from jax import lax
from jax.experimental import pallas as pl
from jax.experimental.pallas import tpu as pltpu
from jax.experimental.pallas import tpu_sc as plsc

N_NODE, N_NET, E_PIN, E_NEAR = 10000, 4000, 20000, 100000
D_IN_NODE, D_IN_NET, D_IN_PIN = 128, 128, 16
H, HP, NT, NL = 64, 16, 8, 2

NODE_P, NET_P = 10240, 4096
EPP, ENP = 20480, 100352
NC, NS, L = 2, 16, 16
FW = 8
CN = 512
CP = 32
GW = (HP + 1) * H

_SC_PARAMS = pltpu.CompilerParams(
    use_tc_tiling_on_sc=False, needs_layout_passes=False)

_NEG = -1e30


def _mm(x, w, b, act, bm, out_dtype=jnp.float32):
    M, K = x.shape
    N = w.shape[1]

    def body(x_ref, w_ref, b_ref, o_ref):
        y = jnp.dot(x_ref[...], w_ref[...],
                    preferred_element_type=jnp.float32) + b_ref[...]
        o_ref[...] = act(y)

    return pl.pallas_call(
        body,
        grid=(M // bm,),
        in_specs=[
            pl.BlockSpec((bm, K), lambda i: (i, 0)),
            pl.BlockSpec((K, N), lambda i: (0, 0)),
            pl.BlockSpec((1, N), lambda i: (0, 0)),
        ],
        out_specs=pl.BlockSpec((bm, N), lambda i: (i, 0)),
        out_shape=jax.ShapeDtypeStruct((M, N), out_dtype),
    )(x, w, b.reshape(1, N))


def _lrelu(y):
    return jnp.where(y > 0, y, 0.01 * y)


def _hp_xs(node, dn3, wp, bp, bm=512):
    M = node.shape[0]

    def body(nd_ref, dn_ref, wp_ref, bp_ref, hp_ref, xs_ref):
        nd = nd_ref[...]
        hp_ref[...] = jnp.maximum(
            jnp.dot(nd, wp_ref[...], preferred_element_type=jnp.float32)
            + bp_ref[...], 0.0)
        d = dn_ref[0] + dn_ref[1]
        xs_ref[...] = nd * lax.rsqrt(jnp.maximum(d, 1.0))

    return pl.pallas_call(
        body,
        grid=(M // bm,),
        in_specs=[
            pl.BlockSpec((bm, H), lambda i: (i, 0)),
            pl.BlockSpec((2, bm, 1), lambda i: (0, i, 0)),
            pl.BlockSpec((H, H), lambda i: (0, 0)),
            pl.BlockSpec((1, H), lambda i: (0, 0)),
        ],
        out_specs=[
            pl.BlockSpec((bm, H), lambda i: (i, 0)),
            pl.BlockSpec((bm, H), lambda i: (i, 0)),
        ],
        out_shape=[
            jax.ShapeDtypeStruct((M, H), jnp.float32),
            jax.ShapeDtypeStruct((M, H), jnp.float32),
        ],
    )(node, dn3, wp, bp.reshape(1, H))


def _net_epilogue(agg_parts, dnt3, gc_w, gc_b, bm=512):
    M = agg_parts.shape[1]

    def body(a_ref, d_ref, w_ref, b_ref, o_ref):
        a = a_ref[0] + a_ref[1]
        d = d_ref[0] + d_ref[1]
        x = a * lax.rsqrt(jnp.maximum(d, 1.0))
        o_ref[...] = jnp.dot(
            x, w_ref[...], preferred_element_type=jnp.float32) + b_ref[...]

    return pl.pallas_call(
        body,
        grid=(M // bm,),
        in_specs=[
            pl.BlockSpec((2, bm, H), lambda i: (0, i, 0)),
            pl.BlockSpec((2, bm, 1), lambda i: (0, i, 0)),
            pl.BlockSpec((H, H), lambda i: (0, 0)),
            pl.BlockSpec((1, H), lambda i: (0, 0)),
        ],
        out_specs=pl.BlockSpec((bm, H), lambda i: (i, 0)),
        out_shape=jax.ShapeDtypeStruct((M, H), jnp.float32),
    )(agg_parts, dnt3, gc_w, gc_b.reshape(1, H))


def _node_epilogue(node, s_parts, m2, dn3, w_self, w_neigh, sage_b, nn_b,
                   bm=512):
    M = node.shape[0]

    def body(nd_ref, s_ref, m_ref, d_ref, ws_ref, wn_ref, sb_ref, nb_ref,
             o_ref):
        nd = nd_ref[...]
        s = s_ref[0] + s_ref[1]
        d = jnp.maximum(d_ref[0] + d_ref[1], 1.0)
        nn_out = s / d + nb_ref[...]
        m = jnp.maximum(jnp.maximum(m_ref[0], m_ref[1]),
                        jnp.maximum(m_ref[2], m_ref[3]))
        m = jnp.where(m > -1e29, m, 0.0)
        sage = (jnp.dot(nd, ws_ref[...], preferred_element_type=jnp.float32)
                + jnp.dot(m, wn_ref[...], preferred_element_type=jnp.float32)
                + sb_ref[...])
        o_ref[...] = jnp.maximum(nn_out, sage)

    return pl.pallas_call(
        body,
        grid=(M // bm,),
        in_specs=[
            pl.BlockSpec((bm, H), lambda i: (i, 0)),
            pl.BlockSpec((2, bm, H), lambda i: (0, i, 0)),
            pl.BlockSpec((4, bm, H), lambda i: (0, i, 0)),
            pl.BlockSpec((2, bm, 1), lambda i: (0, i, 0)),
            pl.BlockSpec((H, H), lambda i: (0, 0)),
            pl.BlockSpec((H, H), lambda i: (0, 0)),
            pl.BlockSpec((1, H), lambda i: (0, 0)),
            pl.BlockSpec((1, H), lambda i: (0, 0)),
        ],
        out_specs=pl.BlockSpec((bm, H), lambda i: (i, 0)),
        out_shape=jax.ShapeDtypeStruct((M, H), jnp.float32),
    )(node, s_parts, m2, dn3, w_self, w_neigh, sage_b.reshape(1, H),
      nn_b.reshape(1, H))


def _mlp(xn, node, o1a, o1b, o1_b, o2_w, o2_b, o3_w, o3_b, bm=512):
    M = xn.shape[0]

    def body(xn_ref, nd_ref, a_ref, b_ref, b1_ref, w2_ref, b2_ref, w3_ref,
             b3_ref, o_ref):
        h = jnp.tanh(
            jnp.dot(xn_ref[...], a_ref[...], preferred_element_type=jnp.float32)
            + jnp.dot(nd_ref[...], b_ref[...],
                      preferred_element_type=jnp.float32)
            + b1_ref[...])
        h = jnp.tanh(
            jnp.dot(h, w2_ref[...], preferred_element_type=jnp.float32)
            + b2_ref[...])
        y = (jnp.dot(h, w3_ref[...], preferred_element_type=jnp.float32)
             + b3_ref[...])
        o_ref[...] = jax.nn.sigmoid(y)

    return pl.pallas_call(
        body,
        grid=(M // bm,),
        in_specs=[
            pl.BlockSpec((bm, D_IN_NODE), lambda i: (i, 0)),
            pl.BlockSpec((bm, H), lambda i: (i, 0)),
            pl.BlockSpec((D_IN_NODE, H), lambda i: (0, 0)),
            pl.BlockSpec((H, H), lambda i: (0, 0)),
            pl.BlockSpec((1, H), lambda i: (0, 0)),
            pl.BlockSpec((H, H), lambda i: (0, 0)),
            pl.BlockSpec((1, H), lambda i: (0, 0)),
            pl.BlockSpec((H, NT), lambda i: (0, 0)),
            pl.BlockSpec((1, NT), lambda i: (0, 0)),
        ],
        out_specs=pl.BlockSpec((bm, NT), lambda i: (i, 0)),
        out_shape=jax.ShapeDtypeStruct((M, NT), jnp.float32),
    )(xn, node, o1a, o1b, o1_b.reshape(1, H), o2_w, o2_b.reshape(1, H),
      o3_w, o3_b.reshape(1, NT))


def _sc_mesh():
    return plsc.VectorSubcoreMesh(core_axis_name="c", subcore_axis_name="s")


def _deg_pass(src, dst):
    ept = EPP // (NC * NS)
    rpn = NODE_P // NS
    rpt = NET_P // NS

    @functools.partial(
        pl.kernel, mesh=_sc_mesh(), compiler_params=_SC_PARAMS,
        out_type=(jax.ShapeDtypeStruct((NC, NODE_P), jnp.float32),
                  jax.ShapeDtypeStruct((NC, NET_P), jnp.float32)),
        scratch_types=[
            pltpu.VMEM((ept,), jnp.int32),
            pltpu.VMEM((ept,), jnp.int32),
            pltpu.VMEM((ept,), jnp.float32),
            pltpu.VMEM((rpn,), jnp.float32),
            pltpu.VMEM_SHARED((NODE_P,), jnp.float32),
            pltpu.VMEM_SHARED((NET_P,), jnp.float32),
        ],
    )
    def k(src_hbm, dst_hbm, dn_hbm, dt_hbm, sidx, didx, ones, zb, accn, acct):
        c = lax.axis_index("c")
        s = lax.axis_index("s")
        zero = jnp.zeros((L,), jnp.float32)

        def zb_body(i, _):
            zb[pl.ds(i * L, L)] = zero
            return 0
        lax.fori_loop(0, rpn // L, zb_body, 0)
        pltpu.sync_copy(zb.at[pl.ds(0, rpn)], accn.at[pl.ds(s * rpn, rpn)])
        pltpu.sync_copy(zb.at[pl.ds(0, rpt)], acct.at[pl.ds(s * rpt, rpt)])
        plsc.subcore_barrier()

        one = jnp.ones((L,), jnp.float32)

        def ones_body(i, _):
            ones[pl.ds(i * L, L)] = one
            return 0
        lax.fori_loop(0, ept // L, ones_body, 0)

        base = (c * NS + s) * ept
        pltpu.sync_copy(src_hbm.at[pl.ds(base, ept)], sidx)
        pltpu.sync_copy(dst_hbm.at[pl.ds(base, ept)], didx)
        pltpu.sync_copy(ones, accn.at[sidx], add=True)
        pltpu.sync_copy(ones, acct.at[didx], add=True)
        plsc.subcore_barrier()
        pltpu.sync_copy(accn.at[pl.ds(s * rpn, rpn)],
                        dn_hbm.at[c, pl.ds(s * rpn, rpn)])
        pltpu.sync_copy(acct.at[pl.ds(s * rpt, rpt)],
                        dt_hbm.at[c, pl.ds(s * rpt, rpt)])

    return k(src, dst)


def _agg_pass(src, dst, xs, zeros2d):
    ept = EPP // (NC * NS)
    rpt = NET_P // NS

    @functools.partial(
        pl.kernel, mesh=_sc_mesh(), compiler_params=_SC_PARAMS,
        out_type=jax.ShapeDtypeStruct((NC, NET_P, H), jnp.float32),
        scratch_types=[
            pltpu.VMEM((ept,), jnp.int32),
            pltpu.VMEM((ept,), jnp.int32),
            pltpu.VMEM((ept, H), jnp.float32),
            pltpu.VMEM_SHARED((NET_P, H), jnp.float32),
            pltpu.SemaphoreType.DMA,
        ],
    )
    def k(src_hbm, dst_hbm, xs_hbm, z_hbm, agg_out, sidx, didx, buf, agg_sh,
          sem0):
        c = lax.axis_index("c")
        s = lax.axis_index("s")
        pltpu.sync_copy(z_hbm.at[pl.ds(s * rpt, rpt)],
                        agg_sh.at[pl.ds(s * rpt, rpt)])
        plsc.subcore_barrier()
        tile_base = (c * NS + s) * ept
        pltpu.sync_copy(src_hbm.at[pl.ds(tile_base, ept)], sidx)
        pltpu.sync_copy(dst_hbm.at[pl.ds(tile_base, ept)], didx)
        pltpu.async_copy(xs_hbm.at[sidx], buf, sem0).wait()
        pltpu.sync_copy(buf, agg_sh.at[didx], add=True)
        plsc.subcore_barrier()
        pltpu.sync_copy(agg_sh.at[pl.ds(s * rpt, rpt)],
                        agg_out.at[c, pl.ds(s * rpt, rpt)])

    return k(src, dst, xs, zeros2d)


def _nnconv_pass(src, dst, gaug, pinw, zeros2d):
    ept = EPP // (NC * NS)
    GC = 16
    npair = ept // (2 * GC)
    rpn = NODE_P // NS

    @functools.partial(
        pl.kernel, mesh=_sc_mesh(), compiler_params=_SC_PARAMS,
        out_type=jax.ShapeDtypeStruct((NC, NODE_P, H), jnp.float32),
        scratch_types=[
            pltpu.VMEM((ept,), jnp.int32),
            pltpu.VMEM((ept,), jnp.int32),
            pltpu.VMEM((ept, HP), jnp.float32),
            pltpu.VMEM((ept, H), jnp.float32),
            pltpu.VMEM((GC, GW), jnp.float32),
            pltpu.VMEM((GC, GW), jnp.float32),
            pltpu.VMEM_SHARED((NODE_P, H), jnp.float32),
            pltpu.SemaphoreType.DMA,
            pltpu.SemaphoreType.DMA,
        ],
    )
    def k(src_hbm, dst_hbm, g_hbm, pin_hbm, z_hbm, s_out,
          sidx, didx, pinb, buf, gr0, gr1, s_sh, sem0, sem1):
        c = lax.axis_index("c")
        s = lax.axis_index("s")
        iota = lax.iota(jnp.int32, L)
        pltpu.sync_copy(z_hbm.at[pl.ds(s * rpn, rpn)],
                        s_sh.at[pl.ds(s * rpn, rpn)])
        plsc.subcore_barrier()

        tile_base = (c * NS + s) * ept
        pltpu.sync_copy(src_hbm.at[pl.ds(tile_base, ept)], sidx)
        pltpu.sync_copy(dst_hbm.at[pl.ds(tile_base, ept)], didx)
        pltpu.sync_copy(pin_hbm.at[pl.ds(tile_base, ept)], pinb)

        def gidx_ref(t):
            return didx.at[pl.ds(t * GC, GC)]

        def edge_body(grbuf, ebase, e, _):
            eg = ebase + e
            pw = pinb[eg, pl.ds(0, HP)]
            accs = [grbuf[e, pl.ds(HP * H + c4 * L, L)]
                    for c4 in range(H // L)]
            for kk in range(HP):
                w = pw[iota * 0 + kk]
                for c4 in range(H // L):
                    accs[c4] = accs[c4] + w * grbuf[
                        e, pl.ds(kk * H + c4 * L, L)]
            for c4 in range(H // L):
                buf[eg, pl.ds(c4 * L, L)] = accs[c4]
            return 0

        pltpu.async_copy(g_hbm.at[gidx_ref(0)], gr0, sem0)

        def round_body(q, _):
            t0 = 2 * q
            pltpu.async_copy(g_hbm.at[gidx_ref(t0 + 1)], gr1, sem1)
            pltpu.make_async_copy(g_hbm.at[gidx_ref(t0)], gr0, sem0).wait()
            lax.fori_loop(0, GC, functools.partial(edge_body, gr0, t0 * GC), 0)

            @pl.when(q < npair - 1)
            def _():
                pltpu.async_copy(g_hbm.at[gidx_ref(t0 + 2)], gr0, sem0)
            pltpu.make_async_copy(g_hbm.at[gidx_ref(t0 + 1)], gr1, sem1).wait()
            lax.fori_loop(0, GC, functools.partial(edge_body, gr1,
                                                   (t0 + 1) * GC), 0)
            return 0
        lax.fori_loop(0, npair, round_body, 0)
        pltpu.sync_copy(buf, s_sh.at[sidx], add=True)
        plsc.subcore_barrier()
        pltpu.sync_copy(s_sh.at[pl.ds(s * rpn, rpn)],
                        s_out.at[c, pl.ds(s * rpn, rpn)])

    return k(src, dst, gaug, pinw, zeros2d)


def _near_pass(hpt, src, dst):
    ept = ENP // 4
    SUP = ept // 2
    CN2 = 224
    nch = SUP // CN2
    AW = NODE_P * FW

    @functools.partial(
        pl.kernel, mesh=_sc_mesh(), compiler_params=_SC_PARAMS,
        out_type=jax.ShapeDtypeStruct((NC, 2, 8, AW), jnp.float32),
        scratch_types=[
            pltpu.VMEM((SUP,), jnp.int32),
            pltpu.VMEM((SUP,), jnp.int32),
            pltpu.VMEM((CN2, FW), jnp.float32),
            pltpu.VMEM((CN2, FW), jnp.float32),
            pltpu.VMEM((AW,), jnp.float32),
            pltpu.SemaphoreType.DMA,
            pltpu.SemaphoreType.DMA,
        ],
    )
    def k(hpt_hbm, src_hbm, dst_hbm, out_hbm, sidx, didx, rows0, rows1, acc,
          sem0, sem1):
        c = lax.axis_index("c")
        s = lax.axis_index("s")
        fc = s % 8
        es = s // 8
        ebase = (c * 2 + es) * ept
        iota = lax.iota(jnp.int32, L)
        half = iota // FW
        lane8 = iota % FW
        swap8 = iota ^ FW

        neg = jnp.full((L,), _NEG, jnp.float32)

        def initbody(i, _):
            acc[pl.ds(i * L, L)] = neg
            return 0
        lax.fori_loop(0, AW // L, initbody, 0)

        def gidx_ref(t):
            return sidx.at[pl.ds(t * CN2, CN2)]

        def pair8(rbuf, cbase, i, _):
            vals = []
            ias = []
            for v in range(2):
                dblk = didx[pl.ds(cbase + (2 * i + v) * L, L)]
                for u in range(8):
                    d1 = dblk[2 * u + half]
                    d2 = dblk[2 * u + (1 - half)]
                    rr = (2 * i + v) * L + 2 * u + half
                    hp2 = plsc.load_gather(rbuf, [rr, lane8])
                    hps = hp2[swap8]
                    vals.append(
                        jnp.where(d1 == d2, jnp.maximum(hp2, hps), hp2))
                    ias.append(d1 * FW + lane8)
            for u in range(16):
                cur = plsc.load_gather(acc, [ias[u]])
                plsc.store_scatter(acc, [ias[u]], jnp.maximum(cur, vals[u]))
            return 0

        for sup in range(2):
            base = ebase + sup * SUP
            pltpu.sync_copy(src_hbm.at[pl.ds(base, SUP)], sidx)
            pltpu.sync_copy(dst_hbm.at[pl.ds(base, SUP)], didx)

            def shiftbody(i, _):
                sidx[pl.ds(i * L, L)] = sidx[pl.ds(i * L, L)] + fc * NODE_P
                return 0
            lax.fori_loop(0, SUP // L, shiftbody, 0)

            pltpu.async_copy(hpt_hbm.at[gidx_ref(0)], rows0, sem0)

            def round_body(q, _):
                t0 = 2 * q
                pltpu.async_copy(hpt_hbm.at[gidx_ref(t0 + 1)], rows1, sem1)
                pltpu.make_async_copy(
                    hpt_hbm.at[gidx_ref(t0)], rows0, sem0).wait()
                lax.fori_loop(0, CN2 // (2 * L),
                              functools.partial(pair8, rows0, t0 * CN2), 0)

                @pl.when(q < nch // 2 - 1)
                def _():
                    pltpu.async_copy(hpt_hbm.at[gidx_ref(t0 + 2)], rows0, sem0)
                pltpu.make_async_copy(
                    hpt_hbm.at[gidx_ref(t0 + 1)], rows1, sem1).wait()
                lax.fori_loop(0, CN2 // (2 * L),
                              functools.partial(pair8, rows1, (t0 + 1) * CN2),
                              0)
                return 0
            lax.fori_loop(0, nch // 2, round_body, 0)

        pltpu.sync_copy(acc, out_hbm.at[c, es, fc])

    return k(hpt, src, dst)


def _pad_rows(x, rows):
    return jnp.pad(x, ((0, rows - x.shape[0]), (0, 0)))


def kernel(in_node_feat, in_net_feat, in_pin_feat, pins_src, pins_dst,
           near_src, near_dst, params):
    p = params
    f32 = jnp.float32

    in_node_p = _pad_rows(in_node_feat.astype(f32), NODE_P)
    in_net_p = _pad_rows(in_net_feat.astype(f32), NET_P)
    in_pin_p = _pad_rows(in_pin_feat.astype(f32), EPP)

    i32 = jnp.int32
    psrc = jnp.concatenate([pins_src.astype(i32),
                            jnp.full((EPP - E_PIN,), N_NODE, i32)])
    pdst = jnp.concatenate([pins_dst.astype(i32),
                            jnp.full((EPP - E_PIN,), N_NET, i32)])
    nsrc = jnp.concatenate([near_src.astype(i32),
                            jnp.zeros((ENP - E_NEAR,), i32)])
    ndst = jnp.concatenate([near_dst.astype(i32),
                            jnp.full((ENP - E_NEAR,), N_NODE, i32)])
    zeros2d = jnp.zeros((NODE_P, H), f32)

    node = _mm(in_node_p, p['node_W'], p['node_b'], _lrelu, 512)
    net = _mm(in_net_p, p['net_W'], p['net_b'], _lrelu, 512)
    pinw = _mm(in_pin_p, p['pin_W'], p['pin_b'], _lrelu, 2048)

    dn_parts, dnt_parts = _deg_pass(psrc, pdst)
    dn3 = dn_parts.reshape(NC, NODE_P, 1)
    dnt3 = dnt_parts.reshape(NC, NET_P, 1)

    for l in range(NL):
        lp = p['layers'][l]
        t = lp['lin2_W'].reshape(HP, H, H)
        waug = jnp.concatenate(
            [t.transpose(1, 0, 2).reshape(H, HP * H),
             lp['lin2_b'].reshape(H, H)], axis=1)

        hp, xs = _hp_xs(node, dn3, lp['sage_Wp'], lp['sage_bp'])
        hpt = hp.reshape(NODE_P, 8, FW).transpose(1, 0, 2).reshape(
            8 * NODE_P, FW)
        gaug = _mm(net, waug, jnp.zeros((GW,), f32), lambda y: y, 512)

        agg_parts = _agg_pass(psrc, pdst, xs, zeros2d)
        s_parts = _nnconv_pass(psrc, pdst, gaug, pinw, zeros2d)
        m_parts = _near_pass(hpt, nsrc, ndst)
        m2 = m_parts.reshape(NC * 2, 8, NODE_P, FW).transpose(
            0, 2, 1, 3).reshape(NC * 2, NODE_P, H)

        net = _net_epilogue(agg_parts, dnt3, lp['gc_W'], lp['gc_b'])
        node = _node_epilogue(node, s_parts, m2, dn3, lp['sage_Wself'],
                              lp['sage_Wneigh'], lp['sage_b'], lp['nn_b'])

    o1a = p['o1_W'][:D_IN_NODE]
    o1b = p['o1_W'][D_IN_NODE:]
    out = _mlp(in_node_p, node, o1a, o1b, p['o1_b'], p['o2_W'], p['o2_b'],
               p['o3_W'], p['o3_b'])
    return out[:N_NODE]


# speedup vs baseline: 1.1759x; 1.0383x over previous
import functools

import jax
import jax.numpy as jnp
from jax import lax
from jax.experimental import pallas as pl
from jax.experimental.pallas import tpu as pltpu
from jax.experimental.pallas import tpu_sc as plsc

N_NODE, N_NET, E_PIN, E_NEAR = 10000, 4000, 20000, 100000
D_IN_NODE, D_IN_NET, D_IN_PIN = 128, 128, 16
H, HP, NT, NL = 64, 16, 8, 2

NODE_P, NET_P = 10240, 4096
EPP, ENP = 20480, 100352
NC, NS, L = 2, 16, 16
FW = 8
CN = 512
CP = 32
GW = (HP + 1) * H

_SC_PARAMS = pltpu.CompilerParams(
    use_tc_tiling_on_sc=False, needs_layout_passes=False)

_NEG = -1e30


def _mm(x, w, b, act, bm, out_dtype=jnp.float32):
    M, K = x.shape
    N = w.shape[1]

    def body(x_ref, w_ref, b_ref, o_ref):
        y = jnp.dot(x_ref[...], w_ref[...],
                    preferred_element_type=jnp.float32) + b_ref[...]
        o_ref[...] = act(y)

    return pl.pallas_call(
        body,
        grid=(M // bm,),
        in_specs=[
            pl.BlockSpec((bm, K), lambda i: (i, 0)),
            pl.BlockSpec((K, N), lambda i: (0, 0)),
            pl.BlockSpec((1, N), lambda i: (0, 0)),
        ],
        out_specs=pl.BlockSpec((bm, N), lambda i: (i, 0)),
        out_shape=jax.ShapeDtypeStruct((M, N), out_dtype),
    )(x, w, b.reshape(1, N))


def _lrelu(y):
    return jnp.where(y > 0, y, 0.01 * y)


def _hp_xs(node, dn3, wp, bp, bm=512):
    M = node.shape[0]

    def body(nd_ref, dn_ref, wp_ref, bp_ref, hp_ref, xs_ref):
        nd = nd_ref[...]
        hp_ref[...] = jnp.maximum(
            jnp.dot(nd, wp_ref[...], preferred_element_type=jnp.float32)
            + bp_ref[...], 0.0)
        d = dn_ref[0] + dn_ref[1]
        xs_ref[...] = nd * lax.rsqrt(jnp.maximum(d, 1.0))

    return pl.pallas_call(
        body,
        grid=(M // bm,),
        in_specs=[
            pl.BlockSpec((bm, H), lambda i: (i, 0)),
            pl.BlockSpec((2, bm, 1), lambda i: (0, i, 0)),
            pl.BlockSpec((H, H), lambda i: (0, 0)),
            pl.BlockSpec((1, H), lambda i: (0, 0)),
        ],
        out_specs=[
            pl.BlockSpec((bm, H), lambda i: (i, 0)),
            pl.BlockSpec((bm, H), lambda i: (i, 0)),
        ],
        out_shape=[
            jax.ShapeDtypeStruct((M, H), jnp.float32),
            jax.ShapeDtypeStruct((M, H), jnp.float32),
        ],
    )(node, dn3, wp, bp.reshape(1, H))


def _net_epilogue(agg_parts, dnt3, gc_w, gc_b, bm=512):
    M = agg_parts.shape[1]

    def body(a_ref, d_ref, w_ref, b_ref, o_ref):
        a = a_ref[0] + a_ref[1]
        d = d_ref[0] + d_ref[1]
        x = a * lax.rsqrt(jnp.maximum(d, 1.0))
        o_ref[...] = jnp.dot(
            x, w_ref[...], preferred_element_type=jnp.float32) + b_ref[...]

    return pl.pallas_call(
        body,
        grid=(M // bm,),
        in_specs=[
            pl.BlockSpec((2, bm, H), lambda i: (0, i, 0)),
            pl.BlockSpec((2, bm, 1), lambda i: (0, i, 0)),
            pl.BlockSpec((H, H), lambda i: (0, 0)),
            pl.BlockSpec((1, H), lambda i: (0, 0)),
        ],
        out_specs=pl.BlockSpec((bm, H), lambda i: (i, 0)),
        out_shape=jax.ShapeDtypeStruct((M, H), jnp.float32),
    )(agg_parts, dnt3, gc_w, gc_b.reshape(1, H))


def _node_epilogue(node, s_parts, m2, dn3, w_self, w_neigh, sage_b, nn_b,
                   bm=512):
    M = node.shape[0]

    def body(nd_ref, s_ref, m_ref, d_ref, ws_ref, wn_ref, sb_ref, nb_ref,
             o_ref):
        nd = nd_ref[...]
        s = s_ref[0] + s_ref[1]
        d = jnp.maximum(d_ref[0] + d_ref[1], 1.0)
        nn_out = s / d + nb_ref[...]
        m = jnp.maximum(jnp.maximum(m_ref[0], m_ref[1]),
                        jnp.maximum(m_ref[2], m_ref[3]))
        m = jnp.where(m > -1e29, m, 0.0)
        sage = (jnp.dot(nd, ws_ref[...], preferred_element_type=jnp.float32)
                + jnp.dot(m, wn_ref[...], preferred_element_type=jnp.float32)
                + sb_ref[...])
        o_ref[...] = jnp.maximum(nn_out, sage)

    return pl.pallas_call(
        body,
        grid=(M // bm,),
        in_specs=[
            pl.BlockSpec((bm, H), lambda i: (i, 0)),
            pl.BlockSpec((2, bm, H), lambda i: (0, i, 0)),
            pl.BlockSpec((4, bm, H), lambda i: (0, i, 0)),
            pl.BlockSpec((2, bm, 1), lambda i: (0, i, 0)),
            pl.BlockSpec((H, H), lambda i: (0, 0)),
            pl.BlockSpec((H, H), lambda i: (0, 0)),
            pl.BlockSpec((1, H), lambda i: (0, 0)),
            pl.BlockSpec((1, H), lambda i: (0, 0)),
        ],
        out_specs=pl.BlockSpec((bm, H), lambda i: (i, 0)),
        out_shape=jax.ShapeDtypeStruct((M, H), jnp.float32),
    )(node, s_parts, m2, dn3, w_self, w_neigh, sage_b.reshape(1, H),
      nn_b.reshape(1, H))


def _mlp(xn, node, o1a, o1b, o1_b, o2_w, o2_b, o3_w, o3_b, bm=512):
    M = xn.shape[0]

    def body(xn_ref, nd_ref, a_ref, b_ref, b1_ref, w2_ref, b2_ref, w3_ref,
             b3_ref, o_ref):
        h = jnp.tanh(
            jnp.dot(xn_ref[...], a_ref[...], preferred_element_type=jnp.float32)
            + jnp.dot(nd_ref[...], b_ref[...],
                      preferred_element_type=jnp.float32)
            + b1_ref[...])
        h = jnp.tanh(
            jnp.dot(h, w2_ref[...], preferred_element_type=jnp.float32)
            + b2_ref[...])
        y = (jnp.dot(h, w3_ref[...], preferred_element_type=jnp.float32)
             + b3_ref[...])
        o_ref[...] = jax.nn.sigmoid(y)

    return pl.pallas_call(
        body,
        grid=(M // bm,),
        in_specs=[
            pl.BlockSpec((bm, D_IN_NODE), lambda i: (i, 0)),
            pl.BlockSpec((bm, H), lambda i: (i, 0)),
            pl.BlockSpec((D_IN_NODE, H), lambda i: (0, 0)),
            pl.BlockSpec((H, H), lambda i: (0, 0)),
            pl.BlockSpec((1, H), lambda i: (0, 0)),
            pl.BlockSpec((H, H), lambda i: (0, 0)),
            pl.BlockSpec((1, H), lambda i: (0, 0)),
            pl.BlockSpec((H, NT), lambda i: (0, 0)),
            pl.BlockSpec((1, NT), lambda i: (0, 0)),
        ],
        out_specs=pl.BlockSpec((bm, NT), lambda i: (i, 0)),
        out_shape=jax.ShapeDtypeStruct((M, NT), jnp.float32),
    )(xn, node, o1a, o1b, o1_b.reshape(1, H), o2_w, o2_b.reshape(1, H),
      o3_w, o3_b.reshape(1, NT))


def _sc_mesh():
    return plsc.VectorSubcoreMesh(core_axis_name="c", subcore_axis_name="s")


def _deg_pass(src, dst):
    ept = EPP // (NC * NS)
    rpn = NODE_P // NS
    rpt = NET_P // NS

    @functools.partial(
        pl.kernel, mesh=_sc_mesh(), compiler_params=_SC_PARAMS,
        out_type=(jax.ShapeDtypeStruct((NC, NODE_P), jnp.float32),
                  jax.ShapeDtypeStruct((NC, NET_P), jnp.float32)),
        scratch_types=[
            pltpu.VMEM((ept,), jnp.int32),
            pltpu.VMEM((ept,), jnp.int32),
            pltpu.VMEM((ept,), jnp.float32),
            pltpu.VMEM((rpn,), jnp.float32),
            pltpu.VMEM_SHARED((NODE_P,), jnp.float32),
            pltpu.VMEM_SHARED((NET_P,), jnp.float32),
        ],
    )
    def k(src_hbm, dst_hbm, dn_hbm, dt_hbm, sidx, didx, ones, zb, accn, acct):
        c = lax.axis_index("c")
        s = lax.axis_index("s")
        zero = jnp.zeros((L,), jnp.float32)

        def zb_body(i, _):
            zb[pl.ds(i * L, L)] = zero
            return 0
        lax.fori_loop(0, rpn // L, zb_body, 0)
        pltpu.sync_copy(zb.at[pl.ds(0, rpn)], accn.at[pl.ds(s * rpn, rpn)])
        pltpu.sync_copy(zb.at[pl.ds(0, rpt)], acct.at[pl.ds(s * rpt, rpt)])
        plsc.subcore_barrier()

        one = jnp.ones((L,), jnp.float32)

        def ones_body(i, _):
            ones[pl.ds(i * L, L)] = one
            return 0
        lax.fori_loop(0, ept // L, ones_body, 0)

        base = (c * NS + s) * ept
        pltpu.sync_copy(src_hbm.at[pl.ds(base, ept)], sidx)
        pltpu.sync_copy(dst_hbm.at[pl.ds(base, ept)], didx)
        pltpu.sync_copy(ones, accn.at[sidx], add=True)
        pltpu.sync_copy(ones, acct.at[didx], add=True)
        plsc.subcore_barrier()
        pltpu.sync_copy(accn.at[pl.ds(s * rpn, rpn)],
                        dn_hbm.at[c, pl.ds(s * rpn, rpn)])
        pltpu.sync_copy(acct.at[pl.ds(s * rpt, rpt)],
                        dt_hbm.at[c, pl.ds(s * rpt, rpt)])

    return k(src, dst)


def _agg_pass(src, dst, xs, zeros2d):
    ept = EPP // (NC * NS)
    rpt = NET_P // NS

    @functools.partial(
        pl.kernel, mesh=_sc_mesh(), compiler_params=_SC_PARAMS,
        out_type=jax.ShapeDtypeStruct((NC, NET_P, H), jnp.float32),
        scratch_types=[
            pltpu.VMEM((ept,), jnp.int32),
            pltpu.VMEM((ept,), jnp.int32),
            pltpu.VMEM((ept, H), jnp.float32),
            pltpu.VMEM_SHARED((NET_P, H), jnp.float32),
            pltpu.SemaphoreType.DMA,
        ],
    )
    def k(src_hbm, dst_hbm, xs_hbm, z_hbm, agg_out, sidx, didx, buf, agg_sh,
          sem0):
        c = lax.axis_index("c")
        s = lax.axis_index("s")
        pltpu.sync_copy(z_hbm.at[pl.ds(s * rpt, rpt)],
                        agg_sh.at[pl.ds(s * rpt, rpt)])
        plsc.subcore_barrier()
        tile_base = (c * NS + s) * ept
        pltpu.sync_copy(src_hbm.at[pl.ds(tile_base, ept)], sidx)
        pltpu.sync_copy(dst_hbm.at[pl.ds(tile_base, ept)], didx)
        pltpu.async_copy(xs_hbm.at[sidx], buf, sem0).wait()
        pltpu.sync_copy(buf, agg_sh.at[didx], add=True)
        plsc.subcore_barrier()
        pltpu.sync_copy(agg_sh.at[pl.ds(s * rpt, rpt)],
                        agg_out.at[c, pl.ds(s * rpt, rpt)])

    return k(src, dst, xs, zeros2d)


def _nnconv_pass(src, dst, gaug, pinw, zeros2d):
    ept = EPP // (NC * NS)
    GC = 16
    npair = ept // (2 * GC)
    rpn = NODE_P // NS

    @functools.partial(
        pl.kernel, mesh=_sc_mesh(), compiler_params=_SC_PARAMS,
        out_type=jax.ShapeDtypeStruct((NC, NODE_P, H), jnp.float32),
        scratch_types=[
            pltpu.VMEM((ept,), jnp.int32),
            pltpu.VMEM((ept,), jnp.int32),
            pltpu.VMEM((ept, HP), jnp.float32),
            pltpu.VMEM((ept, H), jnp.float32),
            pltpu.VMEM((GC, GW), jnp.float32),
            pltpu.VMEM((GC, GW), jnp.float32),
            pltpu.VMEM_SHARED((NODE_P, H), jnp.float32),
            pltpu.SemaphoreType.DMA,
            pltpu.SemaphoreType.DMA,
        ],
    )
    def k(src_hbm, dst_hbm, g_hbm, pin_hbm, z_hbm, s_out,
          sidx, didx, pinb, buf, gr0, gr1, s_sh, sem0, sem1):
        c = lax.axis_index("c")
        s = lax.axis_index("s")
        iota = lax.iota(jnp.int32, L)
        pltpu.sync_copy(z_hbm.at[pl.ds(s * rpn, rpn)],
                        s_sh.at[pl.ds(s * rpn, rpn)])
        plsc.subcore_barrier()

        tile_base = (c * NS + s) * ept
        pltpu.sync_copy(src_hbm.at[pl.ds(tile_base, ept)], sidx)
        pltpu.sync_copy(dst_hbm.at[pl.ds(tile_base, ept)], didx)
        pltpu.sync_copy(pin_hbm.at[pl.ds(tile_base, ept)], pinb)

        def gidx_ref(t):
            return didx.at[pl.ds(t * GC, GC)]

        def edge_body(grbuf, ebase, e, _):
            eg = ebase + e
            pw = pinb[eg, pl.ds(0, HP)]
            accs = [grbuf[e, pl.ds(HP * H + c4 * L, L)]
                    for c4 in range(H // L)]
            for kk in range(HP):
                w = pw[iota * 0 + kk]
                for c4 in range(H // L):
                    accs[c4] = accs[c4] + w * grbuf[
                        e, pl.ds(kk * H + c4 * L, L)]
            for c4 in range(H // L):
                buf[eg, pl.ds(c4 * L, L)] = accs[c4]
            return 0

        pltpu.async_copy(g_hbm.at[gidx_ref(0)], gr0, sem0)

        def round_body(q, _):
            t0 = 2 * q
            pltpu.async_copy(g_hbm.at[gidx_ref(t0 + 1)], gr1, sem1)
            pltpu.make_async_copy(g_hbm.at[gidx_ref(t0)], gr0, sem0).wait()
            lax.fori_loop(0, GC, functools.partial(edge_body, gr0, t0 * GC), 0)

            @pl.when(q < npair - 1)
            def _():
                pltpu.async_copy(g_hbm.at[gidx_ref(t0 + 2)], gr0, sem0)
            pltpu.make_async_copy(g_hbm.at[gidx_ref(t0 + 1)], gr1, sem1).wait()
            lax.fori_loop(0, GC, functools.partial(edge_body, gr1,
                                                   (t0 + 1) * GC), 0)
            return 0
        lax.fori_loop(0, npair, round_body, 0)
        pltpu.sync_copy(buf, s_sh.at[sidx], add=True)
        plsc.subcore_barrier()
        pltpu.sync_copy(s_sh.at[pl.ds(s * rpn, rpn)],
                        s_out.at[c, pl.ds(s * rpn, rpn)])

    return k(src, dst, gaug, pinw, zeros2d)


def _near_pass(hpt, src, dst):
    ept = ENP // 4
    SUP = ept // 2
    CN2 = 448
    nch = SUP // CN2
    AW = NODE_P * FW

    @functools.partial(
        pl.kernel, mesh=_sc_mesh(), compiler_params=_SC_PARAMS,
        out_type=jax.ShapeDtypeStruct((NC, 2, 8, AW), jnp.float32),
        scratch_types=[
            pltpu.VMEM((SUP,), jnp.int32),
            pltpu.VMEM((SUP,), jnp.int32),
            pltpu.VMEM((CN2, FW), jnp.float32),
            pltpu.VMEM((CN2, FW), jnp.float32),
            pltpu.VMEM((AW,), jnp.float32),
            pltpu.SemaphoreType.DMA,
            pltpu.SemaphoreType.DMA,
        ],
    )
    def k(hpt_hbm, src_hbm, dst_hbm, out_hbm, sidx, didx, rows0, rows1, acc,
          sem0, sem1):
        c = lax.axis_index("c")
        s = lax.axis_index("s")
        fc = s % 8
        es = s // 8
        ebase = (c * 2 + es) * ept
        iota = lax.iota(jnp.int32, L)
        half = iota // FW
        lane8 = iota % FW
        swap8 = iota ^ FW

        neg = jnp.full((L,), _NEG, jnp.float32)

        def initbody(i, _):
            acc[pl.ds(i * L, L)] = neg
            return 0
        lax.fori_loop(0, AW // L, initbody, 0)

        def gidx_ref(t):
            return sidx.at[pl.ds(t * CN2, CN2)]

        def pair8(rbuf, cbase, i, _):
            vals = []
            ias = []
            for v in range(2):
                dblk = didx[pl.ds(cbase + (2 * i + v) * L, L)]
                for u in range(8):
                    d1 = dblk[2 * u + half]
                    d2 = dblk[2 * u + (1 - half)]
                    rr = (2 * i + v) * L + 2 * u + half
                    hp2 = plsc.load_gather(rbuf, [rr, lane8])
                    hps = hp2[swap8]
                    vals.append(
                        jnp.where(d1 == d2, jnp.maximum(hp2, hps), hp2))
                    ias.append(d1 * FW + lane8)
            for u in range(16):
                cur = plsc.load_gather(acc, [ias[u]])
                plsc.store_scatter(acc, [ias[u]], jnp.maximum(cur, vals[u]))
            return 0

        for sup in range(2):
            base = ebase + sup * SUP
            pltpu.sync_copy(src_hbm.at[pl.ds(base, SUP)], sidx)
            pltpu.sync_copy(dst_hbm.at[pl.ds(base, SUP)], didx)

            def shiftbody(i, _):
                sidx[pl.ds(i * L, L)] = sidx[pl.ds(i * L, L)] + fc * NODE_P
                return 0
            lax.fori_loop(0, SUP // L, shiftbody, 0)

            pltpu.async_copy(hpt_hbm.at[gidx_ref(0)], rows0, sem0)

            def round_body(q, _):
                t0 = 2 * q
                pltpu.async_copy(hpt_hbm.at[gidx_ref(t0 + 1)], rows1, sem1)
                pltpu.make_async_copy(
                    hpt_hbm.at[gidx_ref(t0)], rows0, sem0).wait()
                lax.fori_loop(0, CN2 // (2 * L),
                              functools.partial(pair8, rows0, t0 * CN2), 0)

                @pl.when(q < nch // 2 - 1)
                def _():
                    pltpu.async_copy(hpt_hbm.at[gidx_ref(t0 + 2)], rows0, sem0)
                pltpu.make_async_copy(
                    hpt_hbm.at[gidx_ref(t0 + 1)], rows1, sem1).wait()
                lax.fori_loop(0, CN2 // (2 * L),
                              functools.partial(pair8, rows1, (t0 + 1) * CN2),
                              0)
                return 0
            lax.fori_loop(0, nch // 2, round_body, 0)

        pltpu.sync_copy(acc, out_hbm.at[c, es, fc])

    return k(hpt, src, dst)


def _pad_rows(x, rows):
    return jnp.pad(x, ((0, rows - x.shape[0]), (0, 0)))


def kernel(in_node_feat, in_net_feat, in_pin_feat, pins_src, pins_dst,
           near_src, near_dst, params):
    p = params
    f32 = jnp.float32

    in_node_p = _pad_rows(in_node_feat.astype(f32), NODE_P)
    in_net_p = _pad_rows(in_net_feat.astype(f32), NET_P)
    in_pin_p = _pad_rows(in_pin_feat.astype(f32), EPP)

    i32 = jnp.int32
    psrc = jnp.concatenate([pins_src.astype(i32),
                            jnp.full((EPP - E_PIN,), N_NODE, i32)])
    pdst = jnp.concatenate([pins_dst.astype(i32),
                            jnp.full((EPP - E_PIN,), N_NET, i32)])
    nsrc = jnp.concatenate([near_src.astype(i32),
                            jnp.zeros((ENP - E_NEAR,), i32)])
    ndst = jnp.concatenate([near_dst.astype(i32),
                            jnp.full((ENP - E_NEAR,), N_NODE, i32)])
    zeros2d = jnp.zeros((NODE_P, H), f32)

    node = _mm(in_node_p, p['node_W'], p['node_b'], _lrelu, 512)
    net = _mm(in_net_p, p['net_W'], p['net_b'], _lrelu, 512)
    pinw = _mm(in_pin_p, p['pin_W'], p['pin_b'], _lrelu, 2048)

    dn_parts, dnt_parts = _deg_pass(psrc, pdst)
    dn3 = dn_parts.reshape(NC, NODE_P, 1)
    dnt3 = dnt_parts.reshape(NC, NET_P, 1)

    for l in range(NL):
        lp = p['layers'][l]
        t = lp['lin2_W'].reshape(HP, H, H)
        waug = jnp.concatenate(
            [t.transpose(1, 0, 2).reshape(H, HP * H),
             lp['lin2_b'].reshape(H, H)], axis=1)

        hp, xs = _hp_xs(node, dn3, lp['sage_Wp'], lp['sage_bp'])
        hpt = hp.reshape(NODE_P, 8, FW).transpose(1, 0, 2).reshape(
            8 * NODE_P, FW)
        gaug = _mm(net, waug, jnp.zeros((GW,), f32), lambda y: y, 512)

        agg_parts = _agg_pass(psrc, pdst, xs, zeros2d)
        s_parts = _nnconv_pass(psrc, pdst, gaug, pinw, zeros2d)
        m_parts = _near_pass(hpt, nsrc, ndst)
        m2 = m_parts.reshape(NC * 2, 8, NODE_P, FW).transpose(
            0, 2, 1, 3).reshape(NC * 2, NODE_P, H)

        net = _net_epilogue(agg_parts, dnt3, lp['gc_W'], lp['gc_b'])
        node = _node_epilogue(node, s_parts, m2, dn3, lp['sage_Wself'],
                              lp['sage_Wneigh'], lp['sage_b'], lp['nn_b'])

    o1a = p['o1_W'][:D_IN_NODE]
    o1b = p['o1_W'][D_IN_NODE:]
    out = _mlp(in_node_p, node, o1a, o1b, p['o1_b'], p['o2_W'], p['o2_b'],
               p['o3_W'], p['o3_b'])
    return out[:N_NODE]


# speedup vs baseline: 1.1867x; 1.0092x over previous
import functools

import jax
import jax.numpy as jnp
from jax import lax
from jax.experimental import pallas as pl
from jax.experimental.pallas import tpu as pltpu
from jax.experimental.pallas import tpu_sc as plsc

N_NODE, N_NET, E_PIN, E_NEAR = 10000, 4000, 20000, 100000
D_IN_NODE, D_IN_NET, D_IN_PIN = 128, 128, 16
H, HP, NT, NL = 64, 16, 8, 2

NODE_P, NET_P = 10240, 4096
EPP, ENP = 20480, 100352
NC, NS, L = 2, 16, 16
FW = 8
CN = 512
CP = 32
GW = (HP + 1) * H

_SC_PARAMS = pltpu.CompilerParams(
    use_tc_tiling_on_sc=False, needs_layout_passes=False)

_NEG = -1e30


def _mm(x, w, b, act, bm, out_dtype=jnp.float32):
    M, K = x.shape
    N = w.shape[1]

    def body(x_ref, w_ref, b_ref, o_ref):
        y = jnp.dot(x_ref[...], w_ref[...],
                    preferred_element_type=jnp.float32) + b_ref[...]
        o_ref[...] = act(y)

    return pl.pallas_call(
        body,
        grid=(M // bm,),
        in_specs=[
            pl.BlockSpec((bm, K), lambda i: (i, 0)),
            pl.BlockSpec((K, N), lambda i: (0, 0)),
            pl.BlockSpec((1, N), lambda i: (0, 0)),
        ],
        out_specs=pl.BlockSpec((bm, N), lambda i: (i, 0)),
        out_shape=jax.ShapeDtypeStruct((M, N), out_dtype),
    )(x, w, b.reshape(1, N))


def _lrelu(y):
    return jnp.where(y > 0, y, 0.01 * y)


def _hp_xs(node, dn3, wp, bp, bm=512):
    M = node.shape[0]

    def body(nd_ref, dn_ref, wp_ref, bp_ref, hp_ref, xs_ref):
        nd = nd_ref[...]
        hp_ref[...] = jnp.maximum(
            jnp.dot(nd, wp_ref[...], preferred_element_type=jnp.float32)
            + bp_ref[...], 0.0)
        d = dn_ref[0] + dn_ref[1]
        xs_ref[...] = nd * lax.rsqrt(jnp.maximum(d, 1.0))

    return pl.pallas_call(
        body,
        grid=(M // bm,),
        in_specs=[
            pl.BlockSpec((bm, H), lambda i: (i, 0)),
            pl.BlockSpec((2, bm, 1), lambda i: (0, i, 0)),
            pl.BlockSpec((H, H), lambda i: (0, 0)),
            pl.BlockSpec((1, H), lambda i: (0, 0)),
        ],
        out_specs=[
            pl.BlockSpec((bm, H), lambda i: (i, 0)),
            pl.BlockSpec((bm, H), lambda i: (i, 0)),
        ],
        out_shape=[
            jax.ShapeDtypeStruct((M, H), jnp.float32),
            jax.ShapeDtypeStruct((M, H), jnp.float32),
        ],
    )(node, dn3, wp, bp.reshape(1, H))


def _net_epilogue(agg_parts, dnt3, gc_w, gc_b, bm=512):
    M = agg_parts.shape[1]

    def body(a_ref, d_ref, w_ref, b_ref, o_ref):
        a = a_ref[0] + a_ref[1]
        d = d_ref[0] + d_ref[1]
        x = a * lax.rsqrt(jnp.maximum(d, 1.0))
        o_ref[...] = jnp.dot(
            x, w_ref[...], preferred_element_type=jnp.float32) + b_ref[...]

    return pl.pallas_call(
        body,
        grid=(M // bm,),
        in_specs=[
            pl.BlockSpec((2, bm, H), lambda i: (0, i, 0)),
            pl.BlockSpec((2, bm, 1), lambda i: (0, i, 0)),
            pl.BlockSpec((H, H), lambda i: (0, 0)),
            pl.BlockSpec((1, H), lambda i: (0, 0)),
        ],
        out_specs=pl.BlockSpec((bm, H), lambda i: (i, 0)),
        out_shape=jax.ShapeDtypeStruct((M, H), jnp.float32),
    )(agg_parts, dnt3, gc_w, gc_b.reshape(1, H))


def _node_epilogue(node, s_parts, m2, dn3, w_self, w_neigh, sage_b, nn_b,
                   bm=512):
    M = node.shape[0]

    def body(nd_ref, s_ref, m_ref, d_ref, ws_ref, wn_ref, sb_ref, nb_ref,
             o_ref):
        nd = nd_ref[...]
        s = s_ref[0] + s_ref[1]
        d = jnp.maximum(d_ref[0] + d_ref[1], 1.0)
        nn_out = s / d + nb_ref[...]
        m = jnp.maximum(jnp.maximum(m_ref[0], m_ref[1]),
                        jnp.maximum(m_ref[2], m_ref[3]))
        m = jnp.where(m > -1e29, m, 0.0)
        sage = (jnp.dot(nd, ws_ref[...], preferred_element_type=jnp.float32)
                + jnp.dot(m, wn_ref[...], preferred_element_type=jnp.float32)
                + sb_ref[...])
        o_ref[...] = jnp.maximum(nn_out, sage)

    return pl.pallas_call(
        body,
        grid=(M // bm,),
        in_specs=[
            pl.BlockSpec((bm, H), lambda i: (i, 0)),
            pl.BlockSpec((2, bm, H), lambda i: (0, i, 0)),
            pl.BlockSpec((4, bm, H), lambda i: (0, i, 0)),
            pl.BlockSpec((2, bm, 1), lambda i: (0, i, 0)),
            pl.BlockSpec((H, H), lambda i: (0, 0)),
            pl.BlockSpec((H, H), lambda i: (0, 0)),
            pl.BlockSpec((1, H), lambda i: (0, 0)),
            pl.BlockSpec((1, H), lambda i: (0, 0)),
        ],
        out_specs=pl.BlockSpec((bm, H), lambda i: (i, 0)),
        out_shape=jax.ShapeDtypeStruct((M, H), jnp.float32),
    )(node, s_parts, m2, dn3, w_self, w_neigh, sage_b.reshape(1, H),
      nn_b.reshape(1, H))


def _mlp(xn, node, o1a, o1b, o1_b, o2_w, o2_b, o3_w, o3_b, bm=512):
    M = xn.shape[0]

    def body(xn_ref, nd_ref, a_ref, b_ref, b1_ref, w2_ref, b2_ref, w3_ref,
             b3_ref, o_ref):
        h = jnp.tanh(
            jnp.dot(xn_ref[...], a_ref[...], preferred_element_type=jnp.float32)
            + jnp.dot(nd_ref[...], b_ref[...],
                      preferred_element_type=jnp.float32)
            + b1_ref[...])
        h = jnp.tanh(
            jnp.dot(h, w2_ref[...], preferred_element_type=jnp.float32)
            + b2_ref[...])
        y = (jnp.dot(h, w3_ref[...], preferred_element_type=jnp.float32)
             + b3_ref[...])
        o_ref[...] = jax.nn.sigmoid(y)

    return pl.pallas_call(
        body,
        grid=(M // bm,),
        in_specs=[
            pl.BlockSpec((bm, D_IN_NODE), lambda i: (i, 0)),
            pl.BlockSpec((bm, H), lambda i: (i, 0)),
            pl.BlockSpec((D_IN_NODE, H), lambda i: (0, 0)),
            pl.BlockSpec((H, H), lambda i: (0, 0)),
            pl.BlockSpec((1, H), lambda i: (0, 0)),
            pl.BlockSpec((H, H), lambda i: (0, 0)),
            pl.BlockSpec((1, H), lambda i: (0, 0)),
            pl.BlockSpec((H, NT), lambda i: (0, 0)),
            pl.BlockSpec((1, NT), lambda i: (0, 0)),
        ],
        out_specs=pl.BlockSpec((bm, NT), lambda i: (i, 0)),
        out_shape=jax.ShapeDtypeStruct((M, NT), jnp.float32),
    )(xn, node, o1a, o1b, o1_b.reshape(1, H), o2_w, o2_b.reshape(1, H),
      o3_w, o3_b.reshape(1, NT))


def _sc_mesh():
    return plsc.VectorSubcoreMesh(core_axis_name="c", subcore_axis_name="s")


def _deg_pass(src, dst):
    ept = EPP // (NC * NS)
    rpn = NODE_P // NS
    rpt = NET_P // NS

    @functools.partial(
        pl.kernel, mesh=_sc_mesh(), compiler_params=_SC_PARAMS,
        out_type=(jax.ShapeDtypeStruct((NC, NODE_P), jnp.float32),
                  jax.ShapeDtypeStruct((NC, NET_P), jnp.float32)),
        scratch_types=[
            pltpu.VMEM((ept,), jnp.int32),
            pltpu.VMEM((ept,), jnp.int32),
            pltpu.VMEM((ept,), jnp.float32),
            pltpu.VMEM((rpn,), jnp.float32),
            pltpu.VMEM_SHARED((NODE_P,), jnp.float32),
            pltpu.VMEM_SHARED((NET_P,), jnp.float32),
        ],
    )
    def k(src_hbm, dst_hbm, dn_hbm, dt_hbm, sidx, didx, ones, zb, accn, acct):
        c = lax.axis_index("c")
        s = lax.axis_index("s")
        zero = jnp.zeros((L,), jnp.float32)

        def zb_body(i, _):
            zb[pl.ds(i * L, L)] = zero
            return 0
        lax.fori_loop(0, rpn // L, zb_body, 0)
        pltpu.sync_copy(zb.at[pl.ds(0, rpn)], accn.at[pl.ds(s * rpn, rpn)])
        pltpu.sync_copy(zb.at[pl.ds(0, rpt)], acct.at[pl.ds(s * rpt, rpt)])
        plsc.subcore_barrier()

        one = jnp.ones((L,), jnp.float32)

        def ones_body(i, _):
            ones[pl.ds(i * L, L)] = one
            return 0
        lax.fori_loop(0, ept // L, ones_body, 0)

        base = (c * NS + s) * ept
        pltpu.sync_copy(src_hbm.at[pl.ds(base, ept)], sidx)
        pltpu.sync_copy(dst_hbm.at[pl.ds(base, ept)], didx)
        pltpu.sync_copy(ones, accn.at[sidx], add=True)
        pltpu.sync_copy(ones, acct.at[didx], add=True)
        plsc.subcore_barrier()
        pltpu.sync_copy(accn.at[pl.ds(s * rpn, rpn)],
                        dn_hbm.at[c, pl.ds(s * rpn, rpn)])
        pltpu.sync_copy(acct.at[pl.ds(s * rpt, rpt)],
                        dt_hbm.at[c, pl.ds(s * rpt, rpt)])

    return k(src, dst)


def _agg_pass(src, dst, xs, zeros2d):
    ept = EPP // (NC * NS)
    rpt = NET_P // NS

    @functools.partial(
        pl.kernel, mesh=_sc_mesh(), compiler_params=_SC_PARAMS,
        out_type=jax.ShapeDtypeStruct((NC, NET_P, H), jnp.float32),
        scratch_types=[
            pltpu.VMEM((ept,), jnp.int32),
            pltpu.VMEM((ept,), jnp.int32),
            pltpu.VMEM((ept, H), jnp.float32),
            pltpu.VMEM_SHARED((NET_P, H), jnp.float32),
            pltpu.SemaphoreType.DMA,
        ],
    )
    def k(src_hbm, dst_hbm, xs_hbm, z_hbm, agg_out, sidx, didx, buf, agg_sh,
          sem0):
        c = lax.axis_index("c")
        s = lax.axis_index("s")
        pltpu.sync_copy(z_hbm.at[pl.ds(s * rpt, rpt)],
                        agg_sh.at[pl.ds(s * rpt, rpt)])
        plsc.subcore_barrier()
        tile_base = (c * NS + s) * ept
        pltpu.sync_copy(src_hbm.at[pl.ds(tile_base, ept)], sidx)
        pltpu.sync_copy(dst_hbm.at[pl.ds(tile_base, ept)], didx)
        pltpu.async_copy(xs_hbm.at[sidx], buf, sem0).wait()
        pltpu.sync_copy(buf, agg_sh.at[didx], add=True)
        plsc.subcore_barrier()
        pltpu.sync_copy(agg_sh.at[pl.ds(s * rpt, rpt)],
                        agg_out.at[c, pl.ds(s * rpt, rpt)])

    return k(src, dst, xs, zeros2d)


def _nnconv_pass(src, dst, gaug, pinw, zeros2d):
    ept = EPP // (NC * NS)
    GC = 16
    npair = ept // (2 * GC)
    rpn = NODE_P // NS

    @functools.partial(
        pl.kernel, mesh=_sc_mesh(), compiler_params=_SC_PARAMS,
        out_type=jax.ShapeDtypeStruct((NC, NODE_P, H), jnp.float32),
        scratch_types=[
            pltpu.VMEM((ept,), jnp.int32),
            pltpu.VMEM((ept,), jnp.int32),
            pltpu.VMEM((ept, HP), jnp.float32),
            pltpu.VMEM((ept, H), jnp.float32),
            pltpu.VMEM((GC, GW), jnp.float32),
            pltpu.VMEM((GC, GW), jnp.float32),
            pltpu.VMEM_SHARED((NODE_P, H), jnp.float32),
            pltpu.SemaphoreType.DMA,
            pltpu.SemaphoreType.DMA,
        ],
    )
    def k(src_hbm, dst_hbm, g_hbm, pin_hbm, z_hbm, s_out,
          sidx, didx, pinb, buf, gr0, gr1, s_sh, sem0, sem1):
        c = lax.axis_index("c")
        s = lax.axis_index("s")
        iota = lax.iota(jnp.int32, L)
        pltpu.sync_copy(z_hbm.at[pl.ds(s * rpn, rpn)],
                        s_sh.at[pl.ds(s * rpn, rpn)])
        plsc.subcore_barrier()

        tile_base = (c * NS + s) * ept
        pltpu.sync_copy(src_hbm.at[pl.ds(tile_base, ept)], sidx)
        pltpu.sync_copy(dst_hbm.at[pl.ds(tile_base, ept)], didx)
        pltpu.sync_copy(pin_hbm.at[pl.ds(tile_base, ept)], pinb)

        def gidx_ref(t):
            return didx.at[pl.ds(t * GC, GC)]

        def edge_body(grbuf, ebase, e, _):
            eg = ebase + e
            pw = pinb[eg, pl.ds(0, HP)]
            accs = [grbuf[e, pl.ds(HP * H + c4 * L, L)]
                    for c4 in range(H // L)]
            for kk in range(HP):
                w = pw[iota * 0 + kk]
                for c4 in range(H // L):
                    accs[c4] = accs[c4] + w * grbuf[
                        e, pl.ds(kk * H + c4 * L, L)]
            for c4 in range(H // L):
                buf[eg, pl.ds(c4 * L, L)] = accs[c4]
            return 0

        pltpu.async_copy(g_hbm.at[gidx_ref(0)], gr0, sem0)

        def round_body(q, _):
            t0 = 2 * q
            pltpu.async_copy(g_hbm.at[gidx_ref(t0 + 1)], gr1, sem1)
            pltpu.make_async_copy(g_hbm.at[gidx_ref(t0)], gr0, sem0).wait()
            lax.fori_loop(0, GC, functools.partial(edge_body, gr0, t0 * GC), 0)

            @pl.when(q < npair - 1)
            def _():
                pltpu.async_copy(g_hbm.at[gidx_ref(t0 + 2)], gr0, sem0)
            pltpu.make_async_copy(g_hbm.at[gidx_ref(t0 + 1)], gr1, sem1).wait()
            lax.fori_loop(0, GC, functools.partial(edge_body, gr1,
                                                   (t0 + 1) * GC), 0)
            return 0
        lax.fori_loop(0, npair, round_body, 0)
        pltpu.sync_copy(buf, s_sh.at[sidx], add=True)
        plsc.subcore_barrier()
        pltpu.sync_copy(s_sh.at[pl.ds(s * rpn, rpn)],
                        s_out.at[c, pl.ds(s * rpn, rpn)])

    return k(src, dst, gaug, pinw, zeros2d)


def _near_pass(hpt, src, dst):
    ept = ENP // 4
    SUP = ept // 2
    CN2 = 896
    nch = SUP // CN2
    AW = NODE_P * FW

    @functools.partial(
        pl.kernel, mesh=_sc_mesh(), compiler_params=_SC_PARAMS,
        out_type=jax.ShapeDtypeStruct((NC, 2, 8, AW), jnp.float32),
        scratch_types=[
            pltpu.VMEM((SUP,), jnp.int32),
            pltpu.VMEM((SUP,), jnp.int32),
            pltpu.VMEM((CN2, FW), jnp.float32),
            pltpu.VMEM((CN2, FW), jnp.float32),
            pltpu.VMEM((AW,), jnp.float32),
            pltpu.SemaphoreType.DMA,
            pltpu.SemaphoreType.DMA,
        ],
    )
    def k(hpt_hbm, src_hbm, dst_hbm, out_hbm, sidx, didx, rows0, rows1, acc,
          sem0, sem1):
        c = lax.axis_index("c")
        s = lax.axis_index("s")
        fc = s % 8
        es = s // 8
        ebase = (c * 2 + es) * ept
        iota = lax.iota(jnp.int32, L)
        half = iota // FW
        lane8 = iota % FW
        swap8 = iota ^ FW

        neg = jnp.full((L,), _NEG, jnp.float32)

        def initbody(i, _):
            acc[pl.ds(i * L, L)] = neg
            return 0
        lax.fori_loop(0, AW // L, initbody, 0)

        def gidx_ref(t):
            return sidx.at[pl.ds(t * CN2, CN2)]

        def pair8(rbuf, cbase, i, _):
            vals = []
            ias = []
            for v in range(2):
                dblk = didx[pl.ds(cbase + (2 * i + v) * L, L)]
                for u in range(8):
                    d1 = dblk[2 * u + half]
                    d2 = dblk[2 * u + (1 - half)]
                    rr = (2 * i + v) * L + 2 * u + half
                    hp2 = plsc.load_gather(rbuf, [rr, lane8])
                    hps = hp2[swap8]
                    vals.append(
                        jnp.where(d1 == d2, jnp.maximum(hp2, hps), hp2))
                    ias.append(d1 * FW + lane8)
            for u in range(16):
                cur = plsc.load_gather(acc, [ias[u]])
                plsc.store_scatter(acc, [ias[u]], jnp.maximum(cur, vals[u]))
            return 0

        for sup in range(2):
            base = ebase + sup * SUP
            pltpu.sync_copy(src_hbm.at[pl.ds(base, SUP)], sidx)
            pltpu.sync_copy(dst_hbm.at[pl.ds(base, SUP)], didx)

            def shiftbody(i, _):
                sidx[pl.ds(i * L, L)] = sidx[pl.ds(i * L, L)] + fc * NODE_P
                return 0
            lax.fori_loop(0, SUP // L, shiftbody, 0)

            pltpu.async_copy(hpt_hbm.at[gidx_ref(0)], rows0, sem0)

            def round_body(q, _):
                t0 = 2 * q
                pltpu.async_copy(hpt_hbm.at[gidx_ref(t0 + 1)], rows1, sem1)
                pltpu.make_async_copy(
                    hpt_hbm.at[gidx_ref(t0)], rows0, sem0).wait()
                lax.fori_loop(0, CN2 // (2 * L),
                              functools.partial(pair8, rows0, t0 * CN2), 0)

                @pl.when(q < nch // 2 - 1)
                def _():
                    pltpu.async_copy(hpt_hbm.at[gidx_ref(t0 + 2)], rows0, sem0)
                pltpu.make_async_copy(
                    hpt_hbm.at[gidx_ref(t0 + 1)], rows1, sem1).wait()
                lax.fori_loop(0, CN2 // (2 * L),
                              functools.partial(pair8, rows1, (t0 + 1) * CN2),
                              0)
                return 0
            lax.fori_loop(0, nch // 2, round_body, 0)

        pltpu.sync_copy(acc, out_hbm.at[c, es, fc])

    return k(hpt, src, dst)


def _pad_rows(x, rows):
    return jnp.pad(x, ((0, rows - x.shape[0]), (0, 0)))


def kernel(in_node_feat, in_net_feat, in_pin_feat, pins_src, pins_dst,
           near_src, near_dst, params):
    p = params
    f32 = jnp.float32

    in_node_p = _pad_rows(in_node_feat.astype(f32), NODE_P)
    in_net_p = _pad_rows(in_net_feat.astype(f32), NET_P)
    in_pin_p = _pad_rows(in_pin_feat.astype(f32), EPP)

    i32 = jnp.int32
    psrc = jnp.concatenate([pins_src.astype(i32),
                            jnp.full((EPP - E_PIN,), N_NODE, i32)])
    pdst = jnp.concatenate([pins_dst.astype(i32),
                            jnp.full((EPP - E_PIN,), N_NET, i32)])
    nsrc = jnp.concatenate([near_src.astype(i32),
                            jnp.zeros((ENP - E_NEAR,), i32)])
    ndst = jnp.concatenate([near_dst.astype(i32),
                            jnp.full((ENP - E_NEAR,), N_NODE, i32)])
    zeros2d = jnp.zeros((NODE_P, H), f32)

    node = _mm(in_node_p, p['node_W'], p['node_b'], _lrelu, 512)
    net = _mm(in_net_p, p['net_W'], p['net_b'], _lrelu, 512)
    pinw = _mm(in_pin_p, p['pin_W'], p['pin_b'], _lrelu, 2048)

    dn_parts, dnt_parts = _deg_pass(psrc, pdst)
    dn3 = dn_parts.reshape(NC, NODE_P, 1)
    dnt3 = dnt_parts.reshape(NC, NET_P, 1)

    for l in range(NL):
        lp = p['layers'][l]
        t = lp['lin2_W'].reshape(HP, H, H)
        waug = jnp.concatenate(
            [t.transpose(1, 0, 2).reshape(H, HP * H),
             lp['lin2_b'].reshape(H, H)], axis=1)

        hp, xs = _hp_xs(node, dn3, lp['sage_Wp'], lp['sage_bp'])
        hpt = hp.reshape(NODE_P, 8, FW).transpose(1, 0, 2).reshape(
            8 * NODE_P, FW)
        gaug = _mm(net, waug, jnp.zeros((GW,), f32), lambda y: y, 512)

        agg_parts = _agg_pass(psrc, pdst, xs, zeros2d)
        s_parts = _nnconv_pass(psrc, pdst, gaug, pinw, zeros2d)
        m_parts = _near_pass(hpt, nsrc, ndst)
        m2 = m_parts.reshape(NC * 2, 8, NODE_P, FW).transpose(
            0, 2, 1, 3).reshape(NC * 2, NODE_P, H)

        net = _net_epilogue(agg_parts, dnt3, lp['gc_W'], lp['gc_b'])
        node = _node_epilogue(node, s_parts, m2, dn3, lp['sage_Wself'],
                              lp['sage_Wneigh'], lp['sage_b'], lp['nn_b'])

    o1a = p['o1_W'][:D_IN_NODE]
    o1b = p['o1_W'][D_IN_NODE:]
    out = _mlp(in_node_p, node, o1a, o1b, p['o1_b'], p['o2_W'], p['o2_b'],
               p['o3_W'], p['o3_b'])
    return out[:N_NODE]
